# Initial kernel scaffold; baseline (speedup 1.0000x reference)
import jax
import jax.numpy as jnp
from jax.experimental import pallas as pl


def kernel(x, edge_index_0, edge_index_1, edge_index_2, W1_0, W1_1, W1_2, b1_0, b1_1, b1_2, W2_0, W2_1, W2_2, b2_0, b2_1, b2_2):
    raise NotImplementedError("write your pallas kernel here")


# speedup vs baseline: 2.4619x; 2.4619x over previous
import functools
import jax
import jax.numpy as jnp
from jax import lax
from jax.experimental import pallas as pl
from jax.experimental.pallas import tpu as pltpu
from jax.experimental.pallas import tpu_sc as plsc

N = 50000
E = 160000
D_IN = 128
D_H = 128
D_OUT = 16
NE = 3

NP = 50176
STRIPE = NP // 16
ER = 1280
EPAD = ER * 128 - E
PADR = NP - N
KB = 8
KBC = 4
BN = 512
NBLK = NP // BN

_mesh = plsc.VectorSubcoreMesh(core_axis_name="c", subcore_axis_name="s")


def _fill_1d(ref, n, val):
    v = jnp.full((16,), val, dtype=jnp.float32)

    def body(i, c):
        ref[pl.ds(i * 16, 16)] = v
        return c

    lax.fori_loop(0, n // 16, body, 0)


def _fill_2d(ref, rows, cols, val):
    v = jnp.full((16,), val, dtype=jnp.float32)

    def body(i, c):
        for j in range(cols // 16):
            ref[i, pl.ds(j * 16, 16)] = v
        return c

    lax.fori_loop(0, rows, body, 0)


@functools.partial(
    pl.kernel,
    out_type=jax.ShapeDtypeStruct((6, NP), jnp.float32),
    mesh=_mesh,
    compiler_params=pltpu.CompilerParams(use_tc_tiling_on_sc=False),
    scratch_types=dict(
        acc0=pltpu.VMEM_SHARED((NP,), jnp.float32),
        acc1=pltpu.VMEM_SHARED((NP,), jnp.float32),
        acc2=pltpu.VMEM_SHARED((NP,), jnp.float32),
        idx_v=pltpu.VMEM((KB, 128), jnp.int32),
        ones_v=pltpu.VMEM((128,), jnp.float32),
        zero_v=pltpu.VMEM((STRIPE,), jnp.float32),
    ),
)
def _deg_kernel(srcs, dsts, deg_out, acc0, acc1, acc2, idx_v, ones_v, zero_v):
    cid = lax.axis_index("c")
    sid = lax.axis_index("s")
    accs = [acc0, acc1, acc2]
    _fill_1d(ones_v, 128, 1.0)
    _fill_1d(zero_v, STRIPE, 0.0)
    for h in range(3):
        pltpu.sync_copy(zero_v, accs[h].at[pl.ds(sid * STRIPE, STRIPE)])
    plsc.subcore_barrier()
    r0 = sid * (ER // 16)
    for h in range(3):
        acc = accs[h]
        for core in range(2):
            t = core * 3 + h
            arr = srcs if t in (0, 2, 4) else dsts
            e = t // 2

            @pl.when(cid == core)
            def _(arr=arr, e=e, acc=acc):
                def body(b, c):
                    pltpu.sync_copy(arr.at[e, pl.ds(r0 + b * KB, KB)], idx_v)
                    for j in range(KB):
                        pltpu.sync_copy(ones_v, acc.at[idx_v.at[j]], add=True)
                    return c

                lax.fori_loop(0, ER // 16 // KB, body, 0)

    plsc.subcore_barrier()
    for h in range(3):
        for core in range(2):
            t = core * 3 + h
            arr_is_src = t in (0, 2, 4)
            e = t // 2
            out_row = 2 * e + (0 if arr_is_src else 1)

            @pl.when(cid == core)
            def _(out_row=out_row, h=h):
                pltpu.sync_copy(
                    accs[h].at[pl.ds(sid * STRIPE, STRIPE)],
                    deg_out.at[out_row, pl.ds(sid * STRIPE, STRIPE)],
                )


@functools.partial(
    pl.kernel,
    out_type=jax.ShapeDtypeStruct((NE, 4, NP, 32), jnp.float32),
    mesh=_mesh,
    compiler_params=pltpu.CompilerParams(use_tc_tiling_on_sc=False),
    scratch_types=dict(
        acc=pltpu.VMEM_SHARED((NP, 32), jnp.float32),
        sidx=pltpu.VMEM((KBC, 128), jnp.int32),
        didx=pltpu.VMEM((KBC, 128), jnp.int32),
        rows=pltpu.VMEM((KBC, 128, 32), jnp.float32),
        zero_v=pltpu.VMEM((98, 32), jnp.float32),
        gsem=pltpu.SemaphoreType.DMA,
    ),
)
def _agg1_kernel(y, srcs, dsts, out, acc, sidx, didx, rows, zero_v, gsem):
    cid = lax.axis_index("c")
    sid = lax.axis_index("s")
    _fill_2d(zero_v, 98, 32, 0.0)
    r0 = sid * (ER // 16)
    for e in range(NE):
        for half in range(2):
            for core in range(2):
                c = core * 2 + half

                @pl.when(cid == core)
                def _(e=e, c=c):
                    for r in range(STRIPE // 98):
                        pltpu.sync_copy(
                            zero_v,
                            acc.at[pl.ds(sid * STRIPE + r * 98, 98)],
                        )
                    plsc.subcore_barrier()

                    def body(b, ca):
                        pltpu.sync_copy(srcs.at[e, pl.ds(r0 + b * KBC, KBC)], sidx)
                        pltpu.sync_copy(dsts.at[e, pl.ds(r0 + b * KBC, KBC)], didx)
                        descs = []
                        for j in range(KBC):
                            descs.append(
                                pltpu.async_copy(
                                    y.at[e, c].at[sidx.at[j]], rows.at[j], gsem
                                )
                            )
                        for d in descs:
                            d.wait()
                        for j in range(KBC):
                            pltpu.sync_copy(
                                rows.at[j], acc.at[didx.at[j]], add=True
                            )
                        return ca

                    lax.fori_loop(0, ER // 16 // KBC, body, 0)
                    plsc.subcore_barrier()
                    pltpu.sync_copy(
                        acc.at[pl.ds(sid * STRIPE, STRIPE)],
                        out.at[e, c, pl.ds(sid * STRIPE, STRIPE)],
                    )
                    plsc.subcore_barrier()


@functools.partial(
    pl.kernel,
    out_type=jax.ShapeDtypeStruct((2, NE, NP, 16), jnp.float32),
    mesh=_mesh,
    compiler_params=pltpu.CompilerParams(use_tc_tiling_on_sc=False),
    scratch_types=dict(
        acc=pltpu.VMEM_SHARED((NP, 16), jnp.float32),
        sidx=pltpu.VMEM((KB, 128), jnp.int32),
        didx=pltpu.VMEM((KB, 128), jnp.int32),
        rows=pltpu.VMEM((KB, 128, 16), jnp.float32),
        zero_v=pltpu.VMEM((784, 16), jnp.float32),
        gsem=pltpu.SemaphoreType.DMA,
    ),
)
def _agg2_kernel(g, srcs, dsts, out, acc, sidx, didx, rows, zero_v, gsem):
    cid = lax.axis_index("c")
    sid = lax.axis_index("s")
    _fill_2d(zero_v, 784, 16, 0.0)
    r0 = cid * (ER // 2) + sid * (ER // 32)
    for e in range(NE):
        for r in range(STRIPE // 784):
            pltpu.sync_copy(
                zero_v, acc.at[pl.ds(sid * STRIPE + r * 784, 784)]
            )
        plsc.subcore_barrier()

        def body(b, ca, e=e):
            pltpu.sync_copy(srcs.at[e, pl.ds(r0 + b * KB, KB)], sidx)
            pltpu.sync_copy(dsts.at[e, pl.ds(r0 + b * KB, KB)], didx)
            descs = []
            for j in range(KB):
                descs.append(
                    pltpu.async_copy(g.at[e].at[sidx.at[j]], rows.at[j], gsem)
                )
            for d in descs:
                d.wait()
            for j in range(KB):
                pltpu.sync_copy(rows.at[j], acc.at[didx.at[j]], add=True)
            return ca

        lax.fori_loop(0, ER // 32 // KB, body, 0)
        plsc.subcore_barrier()
        pltpu.sync_copy(
            acc.at[pl.ds(sid * STRIPE, STRIPE)],
            out.at[cid, e, pl.ds(sid * STRIPE, STRIPE)],
        )
        plsc.subcore_barrier()


def _mm1_body(x_ref, w_ref, deg_ref, out_ref):
    s = lax.rsqrt(jnp.maximum(deg_ref[0, 0], 1.0))
    y = jnp.dot(x_ref[...], w_ref[0], preferred_element_type=jnp.float32)
    y = y * s[:, None]
    for c in range(4):
        out_ref[0, c] = y[:, c * 32:(c + 1) * 32]


def _mm1(x_pad, w1s, deg):
    return pl.pallas_call(
        _mm1_body,
        grid=(NE, NBLK),
        in_specs=[
            pl.BlockSpec((BN, D_IN), lambda e, i: (i, 0)),
            pl.BlockSpec((1, D_IN, D_H), lambda e, i: (e, 0, 0)),
            pl.BlockSpec((1, 1, BN), lambda e, i: (2 * e, 0, i)),
        ],
        out_specs=pl.BlockSpec((1, 4, BN, 32), lambda e, i: (e, 0, i, 0)),
        out_shape=jax.ShapeDtypeStruct((NE, 4, NP, 32), jnp.float32),
    )(x_pad, w1s, deg)


def _mid_body(agg_ref, deg_ref, b1_ref, w2_ref, g_ref):
    i = pl.program_id(0)
    b1sum = b1_ref[0] + b1_ref[1] + b1_ref[2]
    h = jnp.zeros((BN, D_H), jnp.float32)
    for e in range(NE):
        a = jnp.concatenate([agg_ref[e, c] for c in range(4)], axis=1)
        s_in = lax.rsqrt(jnp.maximum(deg_ref[2 * e + 1, 0], 1.0))
        h = h + a * s_in[:, None]
    h = jnp.maximum(h + b1sum[None, :], 0.0)
    row = i * BN + lax.broadcasted_iota(jnp.int32, (BN, 1), 0)
    h = jnp.where(row < N, h, 0.0)
    for e in range(NE):
        s_out = lax.rsqrt(jnp.maximum(deg_ref[2 * e, 0], 1.0))
        g = jnp.dot(h, w2_ref[e], preferred_element_type=jnp.float32)
        g_ref[e] = g * s_out[:, None]


def _mid(agg, deg, b1s, w2s):
    return pl.pallas_call(
        _mid_body,
        grid=(NBLK,),
        in_specs=[
            pl.BlockSpec((NE, 4, BN, 32), lambda i: (0, 0, i, 0)),
            pl.BlockSpec((6, 1, BN), lambda i: (0, 0, i)),
            pl.BlockSpec((NE, D_H), lambda i: (0, 0)),
            pl.BlockSpec((NE, D_H, D_OUT), lambda i: (0, 0, 0)),
        ],
        out_specs=pl.BlockSpec((NE, BN, D_OUT), lambda i: (0, i, 0)),
        out_shape=jax.ShapeDtypeStruct((NE, NP, D_OUT), jnp.float32),
    )(agg, deg, b1s, w2s)


def _fin_body(part_ref, deg_ref, b2_ref, out_ref):
    b2sum = b2_ref[0] + b2_ref[1] + b2_ref[2]
    o = jnp.zeros((BN, D_OUT), jnp.float32)
    for e in range(NE):
        s_in = lax.rsqrt(jnp.maximum(deg_ref[2 * e + 1, 0], 1.0))
        pe = part_ref[0, e] + part_ref[1, e]
        o = o + pe * s_in[:, None]
    out_ref[...] = o + b2sum[None, :]


def _fin(part, deg, b2s):
    return pl.pallas_call(
        _fin_body,
        grid=(NBLK,),
        in_specs=[
            pl.BlockSpec((2, NE, BN, D_OUT), lambda i: (0, 0, i, 0)),
            pl.BlockSpec((6, 1, BN), lambda i: (0, 0, i)),
            pl.BlockSpec((NE, D_OUT), lambda i: (0, 0)),
        ],
        out_specs=pl.BlockSpec((BN, D_OUT), lambda i: (i, 0)),
        out_shape=jax.ShapeDtypeStruct((NP, D_OUT), jnp.float32),
    )(part, deg, b2s)


@jax.jit
def kernel(x, edge_index_0, edge_index_1, edge_index_2,
           W1_0, W1_1, W1_2, b1_0, b1_1, b1_2,
           W2_0, W2_1, W2_2, b2_0, b2_1, b2_2):
    pad = N + (jnp.arange(EPAD, dtype=jnp.int32) % PADR)
    srcs, dsts = [], []
    for ei in (edge_index_0, edge_index_1, edge_index_2):
        e32 = ei.astype(jnp.int32)
        srcs.append(jnp.concatenate([e32[0], pad]).reshape(ER, 128))
        dsts.append(jnp.concatenate([e32[1], pad]).reshape(ER, 128))
    srcs = jnp.stack(srcs)
    dsts = jnp.stack(dsts)
    x_pad = jnp.zeros((NP, D_IN), jnp.float32).at[:N].set(x)
    w1s = jnp.stack([W1_0, W1_1, W1_2])
    b1s = jnp.stack([b1_0, b1_1, b1_2])
    w2s = jnp.stack([W2_0, W2_1, W2_2])
    b2s = jnp.stack([b2_0, b2_1, b2_2])

    deg = _deg_kernel(srcs, dsts)
    deg3 = deg.reshape(6, 1, NP)
    y = _mm1(x_pad, w1s, deg3)
    agg = _agg1_kernel(y, srcs, dsts)
    g = _mid(agg, deg3, b1s, w2s)
    part = _agg2_kernel(g, srcs, dsts)
    out = _fin(part, deg3, b2s)
    return out[:N]


# speedup vs baseline: 2.6675x; 1.0835x over previous
import functools
import jax
import jax.numpy as jnp
from jax import lax
from jax.experimental import pallas as pl
from jax.experimental.pallas import tpu as pltpu
from jax.experimental.pallas import tpu_sc as plsc

N = 50000
E = 160000
D_IN = 128
D_H = 128
D_OUT = 16
NE = 3

NP = 50176
STRIPE = NP // 16
ER = 1280
EPAD = ER * 128 - E
PADR = NP - N
KB = 8
KBC = 4
BN = 512
NBLK = NP // BN

_mesh = plsc.VectorSubcoreMesh(core_axis_name="c", subcore_axis_name="s")


def _fill_1d(ref, n, val):
    v = jnp.full((16,), val, dtype=jnp.float32)

    def body(i, c):
        ref[pl.ds(i * 16, 16)] = v
        return c

    lax.fori_loop(0, n // 16, body, 0)


def _fill_2d(ref, rows, cols, val):
    v = jnp.full((16,), val, dtype=jnp.float32)

    def body(i, c):
        for j in range(cols // 16):
            ref[i, pl.ds(j * 16, 16)] = v
        return c

    lax.fori_loop(0, rows, body, 0)


@functools.partial(
    pl.kernel,
    out_type=jax.ShapeDtypeStruct((6, 1, NP), jnp.float32),
    mesh=_mesh,
    compiler_params=pltpu.CompilerParams(use_tc_tiling_on_sc=False),
    scratch_types=dict(
        acc0=pltpu.VMEM_SHARED((NP,), jnp.float32),
        acc1=pltpu.VMEM_SHARED((NP,), jnp.float32),
        acc2=pltpu.VMEM_SHARED((NP,), jnp.float32),
        idx_v=pltpu.VMEM((KB, 128), jnp.int32),
        ones_v=pltpu.VMEM((128,), jnp.float32),
        zero_v=pltpu.VMEM((STRIPE,), jnp.float32),
    ),
)
def _deg_kernel(srcs, dsts, deg_out, acc0, acc1, acc2, idx_v, ones_v, zero_v):
    cid = lax.axis_index("c")
    sid = lax.axis_index("s")
    accs = [acc0, acc1, acc2]
    _fill_1d(ones_v, 128, 1.0)
    _fill_1d(zero_v, STRIPE, 0.0)
    for h in range(3):
        pltpu.sync_copy(zero_v, accs[h].at[pl.ds(sid * STRIPE, STRIPE)])
    plsc.subcore_barrier()
    r0 = sid * (ER // 16)
    for h in range(3):
        acc = accs[h]
        for core in range(2):
            t = core * 3 + h
            arr = srcs if t in (0, 2, 4) else dsts
            e = t // 2

            @pl.when(cid == core)
            def _(arr=arr, e=e, acc=acc):
                def body(b, c):
                    pltpu.sync_copy(arr.at[e, pl.ds(r0 + b * KB, KB)], idx_v)
                    for j in range(KB):
                        pltpu.sync_copy(ones_v, acc.at[idx_v.at[j]], add=True)
                    return c

                lax.fori_loop(0, ER // 16 // KB, body, 0)

    plsc.subcore_barrier()
    for h in range(3):
        for core in range(2):
            t = core * 3 + h
            arr_is_src = t in (0, 2, 4)
            e = t // 2
            out_row = 2 * e + (0 if arr_is_src else 1)

            @pl.when(cid == core)
            def _(out_row=out_row, h=h):
                pltpu.sync_copy(
                    accs[h].at[pl.ds(sid * STRIPE, STRIPE)],
                    deg_out.at[out_row, 0, pl.ds(sid * STRIPE, STRIPE)],
                )


@functools.partial(
    pl.kernel,
    out_type=jax.ShapeDtypeStruct((NE, 4, NP, 32), jnp.float32),
    mesh=_mesh,
    compiler_params=pltpu.CompilerParams(use_tc_tiling_on_sc=False),
    scratch_types=dict(
        acc=pltpu.VMEM_SHARED((NP, 32), jnp.float32),
        sidx=pltpu.VMEM((KBC, 128), jnp.int32),
        didx=pltpu.VMEM((KBC, 128), jnp.int32),
        rows=pltpu.VMEM((KBC, 128, 32), jnp.float32),
        zero_v=pltpu.VMEM((98, 32), jnp.float32),
        gsem=pltpu.SemaphoreType.DMA,
    ),
)
def _agg1_kernel(y, srcs, dsts, out, acc, sidx, didx, rows, zero_v, gsem):
    cid = lax.axis_index("c")
    sid = lax.axis_index("s")
    _fill_2d(zero_v, 98, 32, 0.0)
    r0 = sid * (ER // 16)
    for e in range(NE):
        for half in range(2):
            for core in range(2):
                c = core * 2 + half

                @pl.when(cid == core)
                def _(e=e, c=c):
                    for r in range(STRIPE // 98):
                        pltpu.sync_copy(
                            zero_v,
                            acc.at[pl.ds(sid * STRIPE + r * 98, 98)],
                        )
                    plsc.subcore_barrier()

                    def body(b, ca):
                        pltpu.sync_copy(srcs.at[e, pl.ds(r0 + b * KBC, KBC)], sidx)
                        pltpu.sync_copy(dsts.at[e, pl.ds(r0 + b * KBC, KBC)], didx)
                        descs = []
                        for j in range(KBC):
                            descs.append(
                                pltpu.async_copy(
                                    y.at[e, c].at[sidx.at[j]], rows.at[j], gsem
                                )
                            )
                        for d in descs:
                            d.wait()
                        for j in range(KBC):
                            pltpu.sync_copy(
                                rows.at[j], acc.at[didx.at[j]], add=True
                            )
                        return ca

                    lax.fori_loop(0, ER // 16 // KBC, body, 0)
                    plsc.subcore_barrier()
                    pltpu.sync_copy(
                        acc.at[pl.ds(sid * STRIPE, STRIPE)],
                        out.at[e, c, pl.ds(sid * STRIPE, STRIPE)],
                    )
                    plsc.subcore_barrier()


@functools.partial(
    pl.kernel,
    out_type=jax.ShapeDtypeStruct((2, NE, NP, 16), jnp.float32),
    mesh=_mesh,
    compiler_params=pltpu.CompilerParams(use_tc_tiling_on_sc=False),
    scratch_types=dict(
        acc=pltpu.VMEM_SHARED((NP, 16), jnp.float32),
        sidx=pltpu.VMEM((KB, 128), jnp.int32),
        didx=pltpu.VMEM((KB, 128), jnp.int32),
        rows=pltpu.VMEM((KB, 128, 16), jnp.float32),
        zero_v=pltpu.VMEM((784, 16), jnp.float32),
        gsem=pltpu.SemaphoreType.DMA,
    ),
)
def _agg2_kernel(g, srcs, dsts, out, acc, sidx, didx, rows, zero_v, gsem):
    cid = lax.axis_index("c")
    sid = lax.axis_index("s")
    _fill_2d(zero_v, 784, 16, 0.0)
    r0 = cid * (ER // 2) + sid * (ER // 32)
    for e in range(NE):
        for r in range(STRIPE // 784):
            pltpu.sync_copy(
                zero_v, acc.at[pl.ds(sid * STRIPE + r * 784, 784)]
            )
        plsc.subcore_barrier()

        def body(b, ca, e=e):
            pltpu.sync_copy(srcs.at[e, pl.ds(r0 + b * KB, KB)], sidx)
            pltpu.sync_copy(dsts.at[e, pl.ds(r0 + b * KB, KB)], didx)
            descs = []
            for j in range(KB):
                descs.append(
                    pltpu.async_copy(g.at[e].at[sidx.at[j]], rows.at[j], gsem)
                )
            for d in descs:
                d.wait()
            for j in range(KB):
                pltpu.sync_copy(rows.at[j], acc.at[didx.at[j]], add=True)
            return ca

        lax.fori_loop(0, ER // 32 // KB, body, 0)
        plsc.subcore_barrier()
        pltpu.sync_copy(
            acc.at[pl.ds(sid * STRIPE, STRIPE)],
            out.at[cid, e, pl.ds(sid * STRIPE, STRIPE)],
        )
        plsc.subcore_barrier()


def _mm1_body(x_ref, w_ref, deg_ref, out_ref):
    y = jnp.dot(x_ref[...], w_ref[...], preferred_element_type=jnp.float32)
    for e in range(NE):
        s = lax.rsqrt(jnp.maximum(deg_ref[2 * e, 0], 1.0))
        ye = y[:, e * D_H:(e + 1) * D_H] * s[:, None]
        for c in range(4):
            out_ref[e, c] = ye[:, c * 32:(c + 1) * 32]


def _mm1(x, w1cat, deg):
    return pl.pallas_call(
        _mm1_body,
        grid=(NBLK,),
        in_specs=[
            pl.BlockSpec((BN, D_IN), lambda i: (i, 0)),
            pl.BlockSpec((D_IN, NE * D_H), lambda i: (0, 0)),
            pl.BlockSpec((6, 1, BN), lambda i: (0, 0, i)),
        ],
        out_specs=pl.BlockSpec((NE, 4, BN, 32), lambda i: (0, 0, i, 0)),
        out_shape=jax.ShapeDtypeStruct((NE, 4, NP, 32), jnp.float32),
    )(x, w1cat, deg)


def _mid_body(agg_ref, deg_ref, b1_ref, w2_ref, g_ref):
    i = pl.program_id(0)
    b1sum = b1_ref[0] + b1_ref[1] + b1_ref[2]
    row = i * BN + lax.broadcasted_iota(jnp.int32, (BN, 1), 0)
    valid = row < N
    s_in = [lax.rsqrt(jnp.maximum(deg_ref[2 * e + 1, 0], 1.0)) for e in range(NE)]
    gs = [jnp.zeros((BN, D_OUT), jnp.float32) for _ in range(NE)]
    for c in range(4):
        hc = jnp.zeros((BN, 32), jnp.float32)
        for e in range(NE):
            hc = hc + agg_ref[e, c] * s_in[e][:, None]
        hc = jnp.maximum(hc + b1sum[None, c * 32:(c + 1) * 32], 0.0)
        hc = jnp.where(valid, hc, 0.0)
        for e in range(NE):
            gs[e] = gs[e] + jnp.dot(
                hc, w2_ref[e, c * 32:(c + 1) * 32, :],
                preferred_element_type=jnp.float32)
    for e in range(NE):
        s_out = lax.rsqrt(jnp.maximum(deg_ref[2 * e, 0], 1.0))
        g_ref[e] = gs[e] * s_out[:, None]


def _mid(agg, deg, b1s, w2s):
    return pl.pallas_call(
        _mid_body,
        grid=(NBLK,),
        in_specs=[
            pl.BlockSpec((NE, 4, BN, 32), lambda i: (0, 0, i, 0)),
            pl.BlockSpec((6, 1, BN), lambda i: (0, 0, i)),
            pl.BlockSpec((NE, D_H), lambda i: (0, 0)),
            pl.BlockSpec((NE, D_H, D_OUT), lambda i: (0, 0, 0)),
        ],
        out_specs=pl.BlockSpec((NE, BN, D_OUT), lambda i: (0, i, 0)),
        out_shape=jax.ShapeDtypeStruct((NE, NP, D_OUT), jnp.float32),
    )(agg, deg, b1s, w2s)


def _fin_body(part_ref, deg_ref, b2_ref, out_ref):
    b2sum = b2_ref[0] + b2_ref[1] + b2_ref[2]
    o = jnp.zeros((BN, D_OUT), jnp.float32)
    for e in range(NE):
        s_in = lax.rsqrt(jnp.maximum(deg_ref[2 * e + 1, 0], 1.0))
        pe = part_ref[0, e] + part_ref[1, e]
        o = o + pe * s_in[:, None]
    out_ref[...] = o + b2sum[None, :]


def _fin(part, deg, b2s):
    return pl.pallas_call(
        _fin_body,
        grid=(NBLK,),
        in_specs=[
            pl.BlockSpec((2, NE, BN, D_OUT), lambda i: (0, 0, i, 0)),
            pl.BlockSpec((6, 1, BN), lambda i: (0, 0, i)),
            pl.BlockSpec((NE, D_OUT), lambda i: (0, 0)),
        ],
        out_specs=pl.BlockSpec((BN, D_OUT), lambda i: (i, 0)),
        out_shape=jax.ShapeDtypeStruct((NP, D_OUT), jnp.float32),
    )(part, deg, b2s)


@jax.jit
def kernel(x, edge_index_0, edge_index_1, edge_index_2,
           W1_0, W1_1, W1_2, b1_0, b1_1, b1_2,
           W2_0, W2_1, W2_2, b2_0, b2_1, b2_2):
    pad = N + (jnp.arange(EPAD, dtype=jnp.int32) % PADR)
    srcs, dsts = [], []
    for ei in (edge_index_0, edge_index_1, edge_index_2):
        e32 = ei.astype(jnp.int32)
        srcs.append(jnp.concatenate([e32[0], pad]).reshape(ER, 128))
        dsts.append(jnp.concatenate([e32[1], pad]).reshape(ER, 128))
    srcs = jnp.stack(srcs)
    dsts = jnp.stack(dsts)
    w1cat = jnp.concatenate([W1_0, W1_1, W1_2], axis=1)
    b1s = jnp.stack([b1_0, b1_1, b1_2])
    w2s = jnp.stack([W2_0, W2_1, W2_2])
    b2s = jnp.stack([b2_0, b2_1, b2_2])

    deg3 = _deg_kernel(srcs, dsts)
    y = _mm1(x, w1cat, deg3)
    agg = _agg1_kernel(y, srcs, dsts)
    g = _mid(agg, deg3, b1s, w2s)
    part = _agg2_kernel(g, srcs, dsts)
    out = _fin(part, deg3, b2s)
    return out[:N]


# speedup vs baseline: 3.7285x; 1.3977x over previous
import functools
import jax
import jax.numpy as jnp
from jax import lax
from jax.experimental import pallas as pl
from jax.experimental.pallas import tpu as pltpu
from jax.experimental.pallas import tpu_sc as plsc

N = 50000
E = 160000
D_IN = 128
D_H = 128
D_OUT = 16
NE = 3

NP = 50176
STRIPE = NP // 16
ER = 1280
EPAD = ER * 128 - E
PADR = NP - N
KB = 8
KBC = 4
BN = 512
NBLK = NP // BN

_mesh = plsc.VectorSubcoreMesh(core_axis_name="c", subcore_axis_name="s")


def _fill_1d(ref, n, val):
    v = jnp.full((16,), val, dtype=jnp.float32)

    def body(i, c):
        ref[pl.ds(i * 16, 16)] = v
        return c

    lax.fori_loop(0, n // 16, body, 0)


def _fill_2d(ref, rows, cols, val):
    v = jnp.full((16,), val, dtype=jnp.float32)

    def body(i, c):
        for j in range(cols // 16):
            ref[i, pl.ds(j * 16, 16)] = v
        return c

    lax.fori_loop(0, rows, body, 0)


@functools.partial(
    pl.kernel,
    out_type=jax.ShapeDtypeStruct((6, 1, NP), jnp.float32),
    mesh=_mesh,
    compiler_params=pltpu.CompilerParams(use_tc_tiling_on_sc=False),
    scratch_types=dict(
        acc0=pltpu.VMEM_SHARED((NP,), jnp.float32),
        acc1=pltpu.VMEM_SHARED((NP,), jnp.float32),
        acc2=pltpu.VMEM_SHARED((NP,), jnp.float32),
        idx_v=pltpu.VMEM((KB, 128), jnp.int32),
        ones_v=pltpu.VMEM((128,), jnp.float32),
        zero_v=pltpu.VMEM((STRIPE,), jnp.float32),
    ),
)
def _deg_kernel(srcs, dsts, deg_out, acc0, acc1, acc2, idx_v, ones_v, zero_v):
    cid = lax.axis_index("c")
    sid = lax.axis_index("s")
    accs = [acc0, acc1, acc2]
    _fill_1d(ones_v, 128, 1.0)
    _fill_1d(zero_v, STRIPE, 0.0)
    for h in range(3):
        pltpu.sync_copy(zero_v, accs[h].at[pl.ds(sid * STRIPE, STRIPE)])
    plsc.subcore_barrier()
    r0 = sid * (ER // 16)
    for h in range(3):
        acc = accs[h]
        for core in range(2):
            t = core * 3 + h
            arr = srcs if t in (0, 2, 4) else dsts
            e = t // 2

            @pl.when(cid == core)
            def _(arr=arr, e=e, acc=acc):
                def body(b, c):
                    pltpu.sync_copy(arr.at[e, pl.ds(r0 + b * KB, KB)], idx_v)
                    for j in range(KB):
                        pltpu.sync_copy(ones_v, acc.at[idx_v.at[j]], add=True)
                    return c

                lax.fori_loop(0, ER // 16 // KB, body, 0)

    plsc.subcore_barrier()
    for h in range(3):
        for core in range(2):
            t = core * 3 + h
            arr_is_src = t in (0, 2, 4)
            e = t // 2
            out_row = 2 * e + (0 if arr_is_src else 1)

            @pl.when(cid == core)
            def _(out_row=out_row, h=h):
                pltpu.sync_copy(
                    accs[h].at[pl.ds(sid * STRIPE, STRIPE)],
                    deg_out.at[out_row, 0, pl.ds(sid * STRIPE, STRIPE)],
                )


@functools.partial(
    pl.kernel,
    out_type=jax.ShapeDtypeStruct((NE, NP, 128), jnp.float32),
    mesh=_mesh,
    compiler_params=pltpu.CompilerParams(use_tc_tiling_on_sc=False),
    scratch_types=dict(
        acc=pltpu.VMEM_SHARED((NP, 32), jnp.float32),
        sidx=pltpu.VMEM((KBC, 128), jnp.int32),
        didx=pltpu.VMEM((KBC, 128), jnp.int32),
        rows=pltpu.VMEM((KBC, 128, 32), jnp.float32),
        zero_v=pltpu.VMEM((98, 32), jnp.float32),
        gsem=pltpu.SemaphoreType.DMA,
    ),
)
def _agg1_kernel(y, srcs, dsts, out, acc, sidx, didx, rows, zero_v, gsem):
    cid = lax.axis_index("c")
    sid = lax.axis_index("s")
    _fill_2d(zero_v, 98, 32, 0.0)
    r0 = sid * (ER // 16)
    for e in range(NE):
        for half in range(2):
            for core in range(2):
                c = core * 2 + half

                @pl.when(cid == core)
                def _(e=e, c=c):
                    for r in range(STRIPE // 98):
                        pltpu.sync_copy(
                            zero_v,
                            acc.at[pl.ds(sid * STRIPE + r * 98, 98)],
                        )
                    plsc.subcore_barrier()

                    def body(b, ca):
                        pltpu.sync_copy(srcs.at[c, e, pl.ds(r0 + b * KBC, KBC)], sidx)
                        pltpu.sync_copy(dsts.at[e, pl.ds(r0 + b * KBC, KBC)], didx)
                        descs = []
                        for j in range(KBC):
                            descs.append(
                                pltpu.async_copy(
                                    y.at[e].at[sidx.at[j]], rows.at[j], gsem
                                )
                            )
                        for d in descs:
                            d.wait()
                        for j in range(KBC):
                            pltpu.sync_copy(
                                rows.at[j], acc.at[didx.at[j]], add=True
                            )
                        return ca

                    lax.fori_loop(0, ER // 16 // KBC, body, 0)
                    plsc.subcore_barrier()
                    pltpu.sync_copy(
                        acc.at[pl.ds(sid * STRIPE, STRIPE)],
                        out.at[e, pl.ds(sid * STRIPE, STRIPE),
                               pl.ds(c * 32, 32)],
                    )
                    plsc.subcore_barrier()


@functools.partial(
    pl.kernel,
    out_type=jax.ShapeDtypeStruct((2, NE, NP, 16), jnp.float32),
    mesh=_mesh,
    compiler_params=pltpu.CompilerParams(use_tc_tiling_on_sc=False),
    scratch_types=dict(
        acc=pltpu.VMEM_SHARED((NP, 16), jnp.float32),
        sidx=pltpu.VMEM((KB, 128), jnp.int32),
        didx=pltpu.VMEM((KB, 128), jnp.int32),
        rows=pltpu.VMEM((KB, 128, 16), jnp.float32),
        zero_v=pltpu.VMEM((784, 16), jnp.float32),
        gsem=pltpu.SemaphoreType.DMA,
    ),
)
def _agg2_kernel(g, srcs, dsts, out, acc, sidx, didx, rows, zero_v, gsem):
    cid = lax.axis_index("c")
    sid = lax.axis_index("s")
    _fill_2d(zero_v, 784, 16, 0.0)
    r0 = cid * (ER // 2) + sid * (ER // 32)
    for e in range(NE):
        for r in range(STRIPE // 784):
            pltpu.sync_copy(
                zero_v, acc.at[pl.ds(sid * STRIPE + r * 784, 784)]
            )
        plsc.subcore_barrier()

        def body(b, ca, e=e):
            pltpu.sync_copy(srcs.at[e, pl.ds(r0 + b * KB, KB)], sidx)
            pltpu.sync_copy(dsts.at[e, pl.ds(r0 + b * KB, KB)], didx)
            descs = []
            for j in range(KB):
                descs.append(
                    pltpu.async_copy(g.at[e].at[sidx.at[j]], rows.at[j], gsem)
                )
            for d in descs:
                d.wait()
            for j in range(KB):
                pltpu.sync_copy(rows.at[j], acc.at[didx.at[j]], add=True)
            return ca

        lax.fori_loop(0, ER // 32 // KB, body, 0)
        plsc.subcore_barrier()
        pltpu.sync_copy(
            acc.at[pl.ds(sid * STRIPE, STRIPE)],
            out.at[cid, e, pl.ds(sid * STRIPE, STRIPE)],
        )
        plsc.subcore_barrier()


def _mm1_body(x_ref, w_ref, deg_ref, out_ref):
    y = jnp.dot(x_ref[...], w_ref[...], preferred_element_type=jnp.float32)
    for e in range(NE):
        s = lax.rsqrt(jnp.maximum(deg_ref[2 * e, 0], 1.0))
        out_ref[e] = y[:, e * D_H:(e + 1) * D_H] * s[:, None]


def _mm1(x, w1cat, deg):
    return pl.pallas_call(
        _mm1_body,
        grid=(NBLK,),
        in_specs=[
            pl.BlockSpec((BN, D_IN), lambda i: (i, 0)),
            pl.BlockSpec((D_IN, NE * D_H), lambda i: (0, 0)),
            pl.BlockSpec((6, 1, BN), lambda i: (0, 0, i)),
        ],
        out_specs=pl.BlockSpec((NE, BN, 128), lambda i: (0, i, 0)),
        out_shape=jax.ShapeDtypeStruct((NE, NP, 128), jnp.float32),
    )(x, w1cat, deg)


def _mid_body(agg_ref, deg_ref, b1_ref, w2_ref, g_ref):
    i = pl.program_id(0)
    b1sum = b1_ref[0] + b1_ref[1] + b1_ref[2]
    row = i * BN + lax.broadcasted_iota(jnp.int32, (BN, 1), 0)
    h = jnp.zeros((BN, D_H), jnp.float32)
    for e in range(NE):
        s_in = lax.rsqrt(jnp.maximum(deg_ref[2 * e + 1, 0], 1.0))
        h = h + agg_ref[e] * s_in[:, None]
    h = jnp.maximum(h + b1sum[None, :], 0.0)
    h = jnp.where(row < N, h, 0.0)
    for e in range(NE):
        s_out = lax.rsqrt(jnp.maximum(deg_ref[2 * e, 0], 1.0))
        g = jnp.dot(h, w2_ref[e], preferred_element_type=jnp.float32)
        g_ref[e] = g * s_out[:, None]


def _mid(agg, deg, b1s, w2s):
    return pl.pallas_call(
        _mid_body,
        grid=(NBLK,),
        in_specs=[
            pl.BlockSpec((NE, BN, 128), lambda i: (0, i, 0)),
            pl.BlockSpec((6, 1, BN), lambda i: (0, 0, i)),
            pl.BlockSpec((NE, D_H), lambda i: (0, 0)),
            pl.BlockSpec((NE, D_H, D_OUT), lambda i: (0, 0, 0)),
        ],
        out_specs=pl.BlockSpec((NE, BN, D_OUT), lambda i: (0, i, 0)),
        out_shape=jax.ShapeDtypeStruct((NE, NP, D_OUT), jnp.float32),
    )(agg, deg, b1s, w2s)


def _fin_body(part_ref, deg_ref, b2_ref, out_ref):
    b2sum = b2_ref[0] + b2_ref[1] + b2_ref[2]
    o = jnp.zeros((BN, D_OUT), jnp.float32)
    for e in range(NE):
        s_in = lax.rsqrt(jnp.maximum(deg_ref[2 * e + 1, 0], 1.0))
        pe = part_ref[0, e] + part_ref[1, e]
        o = o + pe * s_in[:, None]
    out_ref[...] = o + b2sum[None, :]


def _fin(part, deg, b2s):
    return pl.pallas_call(
        _fin_body,
        grid=(NBLK,),
        in_specs=[
            pl.BlockSpec((2, NE, BN, D_OUT), lambda i: (0, 0, i, 0)),
            pl.BlockSpec((6, 1, BN), lambda i: (0, 0, i)),
            pl.BlockSpec((NE, D_OUT), lambda i: (0, 0)),
        ],
        out_specs=pl.BlockSpec((BN, D_OUT), lambda i: (i, 0)),
        out_shape=jax.ShapeDtypeStruct((NP, D_OUT), jnp.float32),
    )(part, deg, b2s)


@jax.jit
def kernel(x, edge_index_0, edge_index_1, edge_index_2,
           W1_0, W1_1, W1_2, b1_0, b1_1, b1_2,
           W2_0, W2_1, W2_2, b2_0, b2_1, b2_2):
    pad = N + (jnp.arange(EPAD, dtype=jnp.int32) % PADR)
    srcs, dsts = [], []
    for ei in (edge_index_0, edge_index_1, edge_index_2):
        e32 = ei.astype(jnp.int32)
        srcs.append(jnp.concatenate([e32[0], pad]).reshape(ER, 128))
        dsts.append(jnp.concatenate([e32[1], pad]).reshape(ER, 128))
    srcs = jnp.stack(srcs)
    dsts = jnp.stack(dsts)
    srcs4 = jnp.stack([srcs * 4 + c for c in range(4)])
    w1cat = jnp.concatenate([W1_0, W1_1, W1_2], axis=1)
    b1s = jnp.stack([b1_0, b1_1, b1_2])
    w2s = jnp.stack([W2_0, W2_1, W2_2])
    b2s = jnp.stack([b2_0, b2_1, b2_2])

    deg3 = _deg_kernel(srcs, dsts)
    y = _mm1(x, w1cat, deg3)
    y4 = y.reshape(NE, NP * 4, 32)
    agg = _agg1_kernel(y4, srcs4, dsts)
    g = _mid(agg, deg3, b1s, w2s)
    part = _agg2_kernel(g, srcs, dsts)
    out = _fin(part, deg3, b2s)
    return out[:N]


# speedup vs baseline: 3.7315x; 1.0008x over previous
import functools
import jax
import jax.numpy as jnp
from jax import lax
from jax.experimental import pallas as pl
from jax.experimental.pallas import tpu as pltpu
from jax.experimental.pallas import tpu_sc as plsc

N = 50000
E = 160000
D_IN = 128
D_H = 128
D_OUT = 16
NE = 3

NP = 50176
STRIPE = NP // 16
ER = 1280
EPAD = ER * 128 - E
PADR = NP - N
EP = ER * 128
WW = 512
BN = 512
NBLK = NP // BN

_mesh = plsc.VectorSubcoreMesh(core_axis_name="c", subcore_axis_name="s")


def _fill_1d(ref, n, val):
    v = jnp.full((16,), val, dtype=jnp.float32)

    def body(i, c):
        ref[pl.ds(i * 16, 16)] = v
        return c

    lax.fori_loop(0, n // 16, body, 0)


def _fill_2d(ref, rows, cols, val):
    v = jnp.full((16,), val, dtype=jnp.float32)

    def body(i, c):
        for j in range(cols // 16):
            ref[i, pl.ds(j * 16, 16)] = v
        return c

    lax.fori_loop(0, rows, body, 0)


@functools.partial(
    pl.kernel,
    out_type=jax.ShapeDtypeStruct((6, 1, NP), jnp.float32),
    mesh=_mesh,
    compiler_params=pltpu.CompilerParams(use_tc_tiling_on_sc=False),
    scratch_types=dict(
        acc0=pltpu.VMEM_SHARED((NP,), jnp.float32),
        acc1=pltpu.VMEM_SHARED((NP,), jnp.float32),
        acc2=pltpu.VMEM_SHARED((NP,), jnp.float32),
        idx_v=pltpu.VMEM((WW,), jnp.int32),
        ones_v=pltpu.VMEM((WW,), jnp.float32),
        zero_v=pltpu.VMEM((STRIPE,), jnp.float32),
    ),
)
def _deg_kernel(srcs, dsts, deg_out, acc0, acc1, acc2, idx_v, ones_v, zero_v):
    cid = lax.axis_index("c")
    sid = lax.axis_index("s")
    accs = [acc0, acc1, acc2]
    _fill_1d(ones_v, WW, 1.0)
    _fill_1d(zero_v, STRIPE, 0.0)
    for h in range(3):
        pltpu.sync_copy(zero_v, accs[h].at[pl.ds(sid * STRIPE, STRIPE)])
    plsc.subcore_barrier()
    r0 = sid * (EP // 16)
    for h in range(3):
        acc = accs[h]
        for core in range(2):
            t = core * 3 + h
            arr = srcs if t in (0, 2, 4) else dsts
            e = t // 2

            @pl.when(cid == core)
            def _(arr=arr, e=e, acc=acc):
                def body(b, c):
                    pltpu.sync_copy(arr.at[e, pl.ds(r0 + b * WW, WW)], idx_v)
                    pltpu.sync_copy(ones_v, acc.at[idx_v], add=True)
                    return c

                lax.fori_loop(0, EP // 16 // WW, body, 0)

    plsc.subcore_barrier()
    for h in range(3):
        for core in range(2):
            t = core * 3 + h
            arr_is_src = t in (0, 2, 4)
            e = t // 2
            out_row = 2 * e + (0 if arr_is_src else 1)

            @pl.when(cid == core)
            def _(out_row=out_row, h=h):
                pltpu.sync_copy(
                    accs[h].at[pl.ds(sid * STRIPE, STRIPE)],
                    deg_out.at[out_row, 0, pl.ds(sid * STRIPE, STRIPE)],
                )


@functools.partial(
    pl.kernel,
    out_type=jax.ShapeDtypeStruct((NE, NP, 128), jnp.float32),
    mesh=_mesh,
    compiler_params=pltpu.CompilerParams(use_tc_tiling_on_sc=False),
    scratch_types=dict(
        acc=pltpu.VMEM_SHARED((NP, 32), jnp.float32),
        sidx=pltpu.VMEM((WW,), jnp.int32),
        didx=pltpu.VMEM((WW,), jnp.int32),
        rows=pltpu.VMEM((WW, 32), jnp.float32),
        zero_v=pltpu.VMEM((98, 32), jnp.float32),
        gsem=pltpu.SemaphoreType.DMA,
    ),
)
def _agg1_kernel(y, srcs, dsts, out, acc, sidx, didx, rows, zero_v, gsem):
    cid = lax.axis_index("c")
    sid = lax.axis_index("s")
    _fill_2d(zero_v, 98, 32, 0.0)
    r0 = sid * (EP // 16)
    for e in range(NE):
        for half in range(2):
            for core in range(2):
                c = core * 2 + half

                @pl.when(cid == core)
                def _(e=e, c=c):
                    for r in range(STRIPE // 98):
                        pltpu.sync_copy(
                            zero_v,
                            acc.at[pl.ds(sid * STRIPE + r * 98, 98)],
                        )
                    plsc.subcore_barrier()

                    def body(b, ca):
                        pltpu.sync_copy(srcs.at[c, e, pl.ds(r0 + b * WW, WW)], sidx)
                        pltpu.sync_copy(dsts.at[e, pl.ds(r0 + b * WW, WW)], didx)
                        pltpu.async_copy(y.at[e].at[sidx], rows, gsem).wait()
                        pltpu.sync_copy(rows, acc.at[didx], add=True)
                        return ca

                    lax.fori_loop(0, EP // 16 // WW, body, 0)
                    plsc.subcore_barrier()
                    pltpu.sync_copy(
                        acc.at[pl.ds(sid * STRIPE, STRIPE)],
                        out.at[e, pl.ds(sid * STRIPE, STRIPE),
                               pl.ds(c * 32, 32)],
                    )
                    plsc.subcore_barrier()


@functools.partial(
    pl.kernel,
    out_type=jax.ShapeDtypeStruct((2, NE, NP, 16), jnp.float32),
    mesh=_mesh,
    compiler_params=pltpu.CompilerParams(use_tc_tiling_on_sc=False),
    scratch_types=dict(
        acc=pltpu.VMEM_SHARED((NP, 16), jnp.float32),
        sidx=pltpu.VMEM((WW,), jnp.int32),
        didx=pltpu.VMEM((WW,), jnp.int32),
        rows=pltpu.VMEM((WW, 16), jnp.float32),
        zero_v=pltpu.VMEM((784, 16), jnp.float32),
        gsem=pltpu.SemaphoreType.DMA,
    ),
)
def _agg2_kernel(g, srcs, dsts, out, acc, sidx, didx, rows, zero_v, gsem):
    cid = lax.axis_index("c")
    sid = lax.axis_index("s")
    _fill_2d(zero_v, 784, 16, 0.0)
    r0 = cid * (EP // 2) + sid * (EP // 32)
    for e in range(NE):
        for r in range(STRIPE // 784):
            pltpu.sync_copy(
                zero_v, acc.at[pl.ds(sid * STRIPE + r * 784, 784)]
            )
        plsc.subcore_barrier()

        def body(b, ca, e=e):
            pltpu.sync_copy(srcs.at[e, pl.ds(r0 + b * WW, WW)], sidx)
            pltpu.sync_copy(dsts.at[e, pl.ds(r0 + b * WW, WW)], didx)
            pltpu.async_copy(g.at[e].at[sidx], rows, gsem).wait()
            pltpu.sync_copy(rows, acc.at[didx], add=True)
            return ca

        lax.fori_loop(0, EP // 32 // WW, body, 0)
        plsc.subcore_barrier()
        pltpu.sync_copy(
            acc.at[pl.ds(sid * STRIPE, STRIPE)],
            out.at[cid, e, pl.ds(sid * STRIPE, STRIPE)],
        )
        plsc.subcore_barrier()


def _mm1_body(x_ref, w_ref, deg_ref, out_ref):
    y = jnp.dot(x_ref[...], w_ref[...], preferred_element_type=jnp.float32)
    for e in range(NE):
        s = lax.rsqrt(jnp.maximum(deg_ref[2 * e, 0], 1.0))
        out_ref[e] = y[:, e * D_H:(e + 1) * D_H] * s[:, None]


def _mm1(x, w1cat, deg):
    return pl.pallas_call(
        _mm1_body,
        grid=(NBLK,),
        in_specs=[
            pl.BlockSpec((BN, D_IN), lambda i: (i, 0)),
            pl.BlockSpec((D_IN, NE * D_H), lambda i: (0, 0)),
            pl.BlockSpec((6, 1, BN), lambda i: (0, 0, i)),
        ],
        out_specs=pl.BlockSpec((NE, BN, 128), lambda i: (0, i, 0)),
        out_shape=jax.ShapeDtypeStruct((NE, NP, 128), jnp.float32),
    )(x, w1cat, deg)


def _mid_body(agg_ref, deg_ref, b1_ref, w2_ref, g_ref):
    i = pl.program_id(0)
    b1sum = b1_ref[0] + b1_ref[1] + b1_ref[2]
    row = i * BN + lax.broadcasted_iota(jnp.int32, (BN, 1), 0)
    h = jnp.zeros((BN, D_H), jnp.float32)
    for e in range(NE):
        s_in = lax.rsqrt(jnp.maximum(deg_ref[2 * e + 1, 0], 1.0))
        h = h + agg_ref[e] * s_in[:, None]
    h = jnp.maximum(h + b1sum[None, :], 0.0)
    h = jnp.where(row < N, h, 0.0)
    for e in range(NE):
        s_out = lax.rsqrt(jnp.maximum(deg_ref[2 * e, 0], 1.0))
        g = jnp.dot(h, w2_ref[e], preferred_element_type=jnp.float32)
        g_ref[e] = g * s_out[:, None]


def _mid(agg, deg, b1s, w2s):
    return pl.pallas_call(
        _mid_body,
        grid=(NBLK,),
        in_specs=[
            pl.BlockSpec((NE, BN, 128), lambda i: (0, i, 0)),
            pl.BlockSpec((6, 1, BN), lambda i: (0, 0, i)),
            pl.BlockSpec((NE, D_H), lambda i: (0, 0)),
            pl.BlockSpec((NE, D_H, D_OUT), lambda i: (0, 0, 0)),
        ],
        out_specs=pl.BlockSpec((NE, BN, D_OUT), lambda i: (0, i, 0)),
        out_shape=jax.ShapeDtypeStruct((NE, NP, D_OUT), jnp.float32),
    )(agg, deg, b1s, w2s)


def _fin_body(part_ref, deg_ref, b2_ref, out_ref):
    b2sum = b2_ref[0] + b2_ref[1] + b2_ref[2]
    o = jnp.zeros((BN, D_OUT), jnp.float32)
    for e in range(NE):
        s_in = lax.rsqrt(jnp.maximum(deg_ref[2 * e + 1, 0], 1.0))
        pe = part_ref[0, e] + part_ref[1, e]
        o = o + pe * s_in[:, None]
    out_ref[...] = o + b2sum[None, :]


def _fin(part, deg, b2s):
    return pl.pallas_call(
        _fin_body,
        grid=(NBLK,),
        in_specs=[
            pl.BlockSpec((2, NE, BN, D_OUT), lambda i: (0, 0, i, 0)),
            pl.BlockSpec((6, 1, BN), lambda i: (0, 0, i)),
            pl.BlockSpec((NE, D_OUT), lambda i: (0, 0)),
        ],
        out_specs=pl.BlockSpec((BN, D_OUT), lambda i: (i, 0)),
        out_shape=jax.ShapeDtypeStruct((NP, D_OUT), jnp.float32),
    )(part, deg, b2s)


@jax.jit
def kernel(x, edge_index_0, edge_index_1, edge_index_2,
           W1_0, W1_1, W1_2, b1_0, b1_1, b1_2,
           W2_0, W2_1, W2_2, b2_0, b2_1, b2_2):
    pad = N + (jnp.arange(EPAD, dtype=jnp.int32) % PADR)
    srcs, dsts = [], []
    for ei in (edge_index_0, edge_index_1, edge_index_2):
        e32 = ei.astype(jnp.int32)
        srcs.append(jnp.concatenate([e32[0], pad]))
        dsts.append(jnp.concatenate([e32[1], pad]))
    srcs = jnp.stack(srcs)
    dsts = jnp.stack(dsts)
    srcs4 = jnp.stack([srcs * 4 + c for c in range(4)])
    w1cat = jnp.concatenate([W1_0, W1_1, W1_2], axis=1)
    b1s = jnp.stack([b1_0, b1_1, b1_2])
    w2s = jnp.stack([W2_0, W2_1, W2_2])
    b2s = jnp.stack([b2_0, b2_1, b2_2])

    deg3 = _deg_kernel(srcs, dsts)
    y = _mm1(x, w1cat, deg3)
    y4 = y.reshape(NE, NP * 4, 32)
    agg = _agg1_kernel(y4, srcs4, dsts)
    g = _mid(agg, deg3, b1s, w2s)
    part = _agg2_kernel(g, srcs, dsts)
    out = _fin(part, deg3, b2s)
    return out[:N]


# speedup vs baseline: 3.9842x; 1.0677x over previous
import functools
import jax
import jax.numpy as jnp
from jax import lax
from jax.experimental import pallas as pl
from jax.experimental.pallas import tpu as pltpu
from jax.experimental.pallas import tpu_sc as plsc

N = 50000
E = 160000
D_IN = 128
D_H = 128
D_OUT = 16
NE = 3

NP = 50176
STRIPE = NP // 16
ER = 1280
EPAD = ER * 128 - E
PADR = NP - N
EP = ER * 128
WW = 512
BN = 512
NBLK = NP // BN

_mesh = plsc.VectorSubcoreMesh(core_axis_name="c", subcore_axis_name="s")


def _fill_1d(ref, n, val):
    v = jnp.full((16,), val, dtype=jnp.float32)

    def body(i, c):
        ref[pl.ds(i * 16, 16)] = v
        return c

    lax.fori_loop(0, n // 16, body, 0)


def _fill_2d(ref, rows, cols, val):
    v = jnp.full((16,), val, dtype=jnp.float32)

    def body(i, c):
        for j in range(cols // 16):
            ref[i, pl.ds(j * 16, 16)] = v
        return c

    lax.fori_loop(0, rows, body, 0)


def _gs_pipeline(src_tab, sidxb, didxb, rows0, rows1, acc, g0, g1, s0, s1, nk):
    pltpu.async_copy(src_tab.at[sidxb.at[0]], rows0, g0)

    def body(b2, ca):
        k0 = 2 * b2
        k1 = k0 + 1
        pltpu.make_async_copy(src_tab.at[sidxb.at[k0]], rows0, g0).wait()

        @pl.when(b2 > 0)
        def _():
            pltpu.make_async_copy(rows1, acc.at[didxb.at[k1]], s1).wait()

        pltpu.async_copy(src_tab.at[sidxb.at[k1]], rows1, g1)
        pltpu.async_copy(rows0, acc.at[didxb.at[k0]], s0, add=True)

        @pl.when(b2 < nk // 2 - 1)
        def _():
            pltpu.make_async_copy(rows0, acc.at[didxb.at[k0]], s0).wait()
            pltpu.async_copy(src_tab.at[sidxb.at[k0 + 2]], rows0, g0)

        pltpu.make_async_copy(src_tab.at[sidxb.at[k1]], rows1, g1).wait()
        pltpu.async_copy(rows1, acc.at[didxb.at[k1]], s1, add=True)
        return ca

    lax.fori_loop(0, nk // 2, body, 0)
    pltpu.make_async_copy(rows0, acc.at[didxb.at[0]], s0).wait()
    pltpu.make_async_copy(rows1, acc.at[didxb.at[0]], s1).wait()


@functools.partial(
    pl.kernel,
    out_type=jax.ShapeDtypeStruct((6, 1, NP), jnp.float32),
    mesh=_mesh,
    compiler_params=pltpu.CompilerParams(use_tc_tiling_on_sc=False),
    scratch_types=dict(
        acc0=pltpu.VMEM_SHARED((NP,), jnp.float32),
        acc1=pltpu.VMEM_SHARED((NP,), jnp.float32),
        acc2=pltpu.VMEM_SHARED((NP,), jnp.float32),
        idx_v=pltpu.VMEM((WW,), jnp.int32),
        ones_v=pltpu.VMEM((WW,), jnp.float32),
        zero_v=pltpu.VMEM((STRIPE,), jnp.float32),
    ),
)
def _deg_kernel(srcs, dsts, deg_out, acc0, acc1, acc2, idx_v, ones_v, zero_v):
    cid = lax.axis_index("c")
    sid = lax.axis_index("s")
    accs = [acc0, acc1, acc2]
    _fill_1d(ones_v, WW, 1.0)
    _fill_1d(zero_v, STRIPE, 0.0)
    for h in range(3):
        pltpu.sync_copy(zero_v, accs[h].at[pl.ds(sid * STRIPE, STRIPE)])
    plsc.subcore_barrier()
    r0 = sid * (EP // 16)
    for h in range(3):
        acc = accs[h]
        for core in range(2):
            t = core * 3 + h
            arr = srcs if t in (0, 2, 4) else dsts
            e = t // 2

            @pl.when(cid == core)
            def _(arr=arr, e=e, acc=acc):
                def body(b, c):
                    pltpu.sync_copy(arr.at[e, pl.ds(r0 + b * WW, WW)], idx_v)
                    pltpu.sync_copy(ones_v, acc.at[idx_v], add=True)
                    return c

                lax.fori_loop(0, EP // 16 // WW, body, 0)

    plsc.subcore_barrier()
    for h in range(3):
        for core in range(2):
            t = core * 3 + h
            arr_is_src = t in (0, 2, 4)
            e = t // 2
            out_row = 2 * e + (0 if arr_is_src else 1)

            @pl.when(cid == core)
            def _(out_row=out_row, h=h):
                pltpu.sync_copy(
                    accs[h].at[pl.ds(sid * STRIPE, STRIPE)],
                    deg_out.at[out_row, 0, pl.ds(sid * STRIPE, STRIPE)],
                )


@functools.partial(
    pl.kernel,
    out_type=jax.ShapeDtypeStruct((NE, NP, 128), jnp.float32),
    mesh=_mesh,
    compiler_params=pltpu.CompilerParams(use_tc_tiling_on_sc=False),
    scratch_types=dict(
        acc=pltpu.VMEM_SHARED((NP, 32), jnp.float32),
        sidxb=pltpu.VMEM((20, 128), jnp.int32),
        didxb=pltpu.VMEM((20, 128), jnp.int32),
        rows0=pltpu.VMEM((128, 32), jnp.float32),
        rows1=pltpu.VMEM((128, 32), jnp.float32),
        zero_v=pltpu.VMEM((98, 32), jnp.float32),
        g0=pltpu.SemaphoreType.DMA,
        g1=pltpu.SemaphoreType.DMA,
        s0=pltpu.SemaphoreType.DMA,
        s1=pltpu.SemaphoreType.DMA,
    ),
)
def _agg1_kernel(y, srcs, dsts, out, acc, sidxb, didxb, rows0, rows1, zero_v,
                 g0, g1, s0, s1):
    cid = lax.axis_index("c")
    sid = lax.axis_index("s")
    _fill_2d(zero_v, 98, 32, 0.0)
    r0 = sid * 80
    for e in range(NE):
        for half in range(2):
            for core in range(2):
                c = core * 2 + half

                @pl.when(cid == core)
                def _(e=e, c=c):
                    for r in range(STRIPE // 98):
                        pltpu.sync_copy(
                            zero_v,
                            acc.at[pl.ds(sid * STRIPE + r * 98, 98)],
                        )
                    plsc.subcore_barrier()

                    for q in range(4):
                        pltpu.sync_copy(
                            srcs.at[c, e, pl.ds(r0 + q * 20, 20)], sidxb)
                        pltpu.sync_copy(
                            dsts.at[e, pl.ds(r0 + q * 20, 20)], didxb)
                        _gs_pipeline(y.at[e], sidxb, didxb, rows0, rows1,
                                     acc, g0, g1, s0, s1, 20)
                    plsc.subcore_barrier()
                    pltpu.sync_copy(
                        acc.at[pl.ds(sid * STRIPE, STRIPE)],
                        out.at[e, pl.ds(sid * STRIPE, STRIPE),
                               pl.ds(c * 32, 32)],
                    )
                    plsc.subcore_barrier()


@functools.partial(
    pl.kernel,
    out_type=jax.ShapeDtypeStruct((2, NE, NP, 16), jnp.float32),
    mesh=_mesh,
    compiler_params=pltpu.CompilerParams(use_tc_tiling_on_sc=False),
    scratch_types=dict(
        acc=pltpu.VMEM_SHARED((NP, 16), jnp.float32),
        sidxb=pltpu.VMEM((40, 128), jnp.int32),
        didxb=pltpu.VMEM((40, 128), jnp.int32),
        rows0=pltpu.VMEM((128, 16), jnp.float32),
        rows1=pltpu.VMEM((128, 16), jnp.float32),
        zero_v=pltpu.VMEM((196, 16), jnp.float32),
        g0=pltpu.SemaphoreType.DMA,
        g1=pltpu.SemaphoreType.DMA,
        s0=pltpu.SemaphoreType.DMA,
        s1=pltpu.SemaphoreType.DMA,
    ),
)
def _agg2_kernel(g, srcs, dsts, out, acc, sidxb, didxb, rows0, rows1, zero_v,
                 g0, g1, s0, s1):
    cid = lax.axis_index("c")
    sid = lax.axis_index("s")
    _fill_2d(zero_v, 196, 16, 0.0)
    r0 = cid * (ER // 2) + sid * 40
    for e in range(NE):
        for r in range(STRIPE // 196):
            pltpu.sync_copy(
                zero_v, acc.at[pl.ds(sid * STRIPE + r * 196, 196)]
            )
        plsc.subcore_barrier()
        pltpu.sync_copy(srcs.at[e, pl.ds(r0, 40)], sidxb)
        pltpu.sync_copy(dsts.at[e, pl.ds(r0, 40)], didxb)
        _gs_pipeline(g.at[e], sidxb, didxb, rows0, rows1, acc,
                     g0, g1, s0, s1, 40)
        plsc.subcore_barrier()
        pltpu.sync_copy(
            acc.at[pl.ds(sid * STRIPE, STRIPE)],
            out.at[cid, e, pl.ds(sid * STRIPE, STRIPE)],
        )
        plsc.subcore_barrier()


def _mm1_body(x_ref, w_ref, deg_ref, out_ref):
    y = jnp.dot(x_ref[...], w_ref[...], preferred_element_type=jnp.float32)
    for e in range(NE):
        s = lax.rsqrt(jnp.maximum(deg_ref[2 * e, 0], 1.0))
        out_ref[e] = y[:, e * D_H:(e + 1) * D_H] * s[:, None]


def _mm1(x, w1cat, deg):
    return pl.pallas_call(
        _mm1_body,
        grid=(NBLK,),
        in_specs=[
            pl.BlockSpec((BN, D_IN), lambda i: (i, 0)),
            pl.BlockSpec((D_IN, NE * D_H), lambda i: (0, 0)),
            pl.BlockSpec((6, 1, BN), lambda i: (0, 0, i)),
        ],
        out_specs=pl.BlockSpec((NE, BN, 128), lambda i: (0, i, 0)),
        out_shape=jax.ShapeDtypeStruct((NE, NP, 128), jnp.float32),
    )(x, w1cat, deg)


def _mid_body(agg_ref, deg_ref, b1_ref, w2_ref, g_ref):
    i = pl.program_id(0)
    b1sum = b1_ref[0] + b1_ref[1] + b1_ref[2]
    row = i * BN + lax.broadcasted_iota(jnp.int32, (BN, 1), 0)
    h = jnp.zeros((BN, D_H), jnp.float32)
    for e in range(NE):
        s_in = lax.rsqrt(jnp.maximum(deg_ref[2 * e + 1, 0], 1.0))
        h = h + agg_ref[e] * s_in[:, None]
    h = jnp.maximum(h + b1sum[None, :], 0.0)
    h = jnp.where(row < N, h, 0.0)
    for e in range(NE):
        s_out = lax.rsqrt(jnp.maximum(deg_ref[2 * e, 0], 1.0))
        g = jnp.dot(h, w2_ref[e], preferred_element_type=jnp.float32)
        g_ref[e] = g * s_out[:, None]


def _mid(agg, deg, b1s, w2s):
    return pl.pallas_call(
        _mid_body,
        grid=(NBLK,),
        in_specs=[
            pl.BlockSpec((NE, BN, 128), lambda i: (0, i, 0)),
            pl.BlockSpec((6, 1, BN), lambda i: (0, 0, i)),
            pl.BlockSpec((NE, D_H), lambda i: (0, 0)),
            pl.BlockSpec((NE, D_H, D_OUT), lambda i: (0, 0, 0)),
        ],
        out_specs=pl.BlockSpec((NE, BN, D_OUT), lambda i: (0, i, 0)),
        out_shape=jax.ShapeDtypeStruct((NE, NP, D_OUT), jnp.float32),
    )(agg, deg, b1s, w2s)


def _fin_body(part_ref, deg_ref, b2_ref, out_ref):
    b2sum = b2_ref[0] + b2_ref[1] + b2_ref[2]
    o = jnp.zeros((BN, D_OUT), jnp.float32)
    for e in range(NE):
        s_in = lax.rsqrt(jnp.maximum(deg_ref[2 * e + 1, 0], 1.0))
        pe = part_ref[0, e] + part_ref[1, e]
        o = o + pe * s_in[:, None]
    out_ref[...] = o + b2sum[None, :]


def _fin(part, deg, b2s):
    return pl.pallas_call(
        _fin_body,
        grid=(NBLK,),
        in_specs=[
            pl.BlockSpec((2, NE, BN, D_OUT), lambda i: (0, 0, i, 0)),
            pl.BlockSpec((6, 1, BN), lambda i: (0, 0, i)),
            pl.BlockSpec((NE, D_OUT), lambda i: (0, 0)),
        ],
        out_specs=pl.BlockSpec((BN, D_OUT), lambda i: (i, 0)),
        out_shape=jax.ShapeDtypeStruct((NP, D_OUT), jnp.float32),
    )(part, deg, b2s)


@jax.jit
def kernel(x, edge_index_0, edge_index_1, edge_index_2,
           W1_0, W1_1, W1_2, b1_0, b1_1, b1_2,
           W2_0, W2_1, W2_2, b2_0, b2_1, b2_2):
    pad = N + (jnp.arange(EPAD, dtype=jnp.int32) % PADR)
    srcs, dsts = [], []
    for ei in (edge_index_0, edge_index_1, edge_index_2):
        e32 = ei.astype(jnp.int32)
        srcs.append(jnp.concatenate([e32[0], pad]))
        dsts.append(jnp.concatenate([e32[1], pad]))
    srcs = jnp.stack(srcs)
    dsts = jnp.stack(dsts)
    srcs2d = srcs.reshape(NE, ER, 128)
    dsts2d = dsts.reshape(NE, ER, 128)
    srcs4 = jnp.stack([srcs2d * 4 + c for c in range(4)])
    w1cat = jnp.concatenate([W1_0, W1_1, W1_2], axis=1)
    b1s = jnp.stack([b1_0, b1_1, b1_2])
    w2s = jnp.stack([W2_0, W2_1, W2_2])
    b2s = jnp.stack([b2_0, b2_1, b2_2])

    deg3 = _deg_kernel(srcs, dsts)
    y = _mm1(x, w1cat, deg3)
    y4 = y.reshape(NE, NP * 4, 32)
    agg = _agg1_kernel(y4, srcs4, dsts2d)
    g = _mid(agg, deg3, b1s, w2s)
    part = _agg2_kernel(g, srcs2d, dsts2d)
    out = _fin(part, deg3, b2s)
    return out[:N]


# speedup vs baseline: 4.4431x; 1.1152x over previous
import functools
import jax
import jax.numpy as jnp
from jax import lax
from jax.experimental import pallas as pl
from jax.experimental.pallas import tpu as pltpu
from jax.experimental.pallas import tpu_sc as plsc

N = 50000
E = 160000
D_IN = 128
D_H = 128
D_OUT = 16
NE = 3

NP = 50176
STRIPE = NP // 16
ER = 1280
EPAD = ER * 128 - E
PADR = NP - N
EP = ER * 128
WW = 512
BN = 512
NBLK = NP // BN

_mesh = plsc.VectorSubcoreMesh(core_axis_name="c", subcore_axis_name="s")


def _fill_1d(ref, n, val):
    v = jnp.full((16,), val, dtype=jnp.float32)

    def body(i, c):
        ref[pl.ds(i * 16, 16)] = v
        return c

    lax.fori_loop(0, n // 16, body, 0)


def _fill_2d(ref, rows, cols, val):
    v = jnp.full((16,), val, dtype=jnp.float32)

    def body(i, c):
        for j in range(cols // 16):
            ref[i, pl.ds(j * 16, 16)] = v
        return c

    lax.fori_loop(0, rows, body, 0)


def _gs_pipeline(src_tab, sidxb, didxb, rows0, rows1, acc, g0, g1, s0, s1, nk):
    pltpu.async_copy(src_tab.at[sidxb.at[0]], rows0, g0)

    def body(b2, ca):
        k0 = 2 * b2
        k1 = k0 + 1
        pltpu.make_async_copy(src_tab.at[sidxb.at[k0]], rows0, g0).wait()

        @pl.when(b2 > 0)
        def _():
            pltpu.make_async_copy(rows1, acc.at[didxb.at[k1]], s1).wait()

        pltpu.async_copy(src_tab.at[sidxb.at[k1]], rows1, g1)
        pltpu.async_copy(rows0, acc.at[didxb.at[k0]], s0, add=True)

        @pl.when(b2 < nk // 2 - 1)
        def _():
            pltpu.make_async_copy(rows0, acc.at[didxb.at[k0]], s0).wait()
            pltpu.async_copy(src_tab.at[sidxb.at[k0 + 2]], rows0, g0)

        pltpu.make_async_copy(src_tab.at[sidxb.at[k1]], rows1, g1).wait()
        pltpu.async_copy(rows1, acc.at[didxb.at[k1]], s1, add=True)
        return ca

    lax.fori_loop(0, nk // 2, body, 0)
    pltpu.make_async_copy(rows0, acc.at[didxb.at[0]], s0).wait()
    pltpu.make_async_copy(rows1, acc.at[didxb.at[0]], s1).wait()


@functools.partial(
    pl.kernel,
    out_type=jax.ShapeDtypeStruct((6, 1, NP), jnp.float32),
    mesh=_mesh,
    compiler_params=pltpu.CompilerParams(use_tc_tiling_on_sc=False),
    scratch_types=dict(
        acc0=pltpu.VMEM_SHARED((NP,), jnp.float32),
        acc1=pltpu.VMEM_SHARED((NP,), jnp.float32),
        acc2=pltpu.VMEM_SHARED((NP,), jnp.float32),
        idx_v=pltpu.VMEM((WW,), jnp.int32),
        ones_v=pltpu.VMEM((WW,), jnp.float32),
        zero_v=pltpu.VMEM((STRIPE,), jnp.float32),
    ),
)
def _deg_kernel(srcs, dsts, deg_out, acc0, acc1, acc2, idx_v, ones_v, zero_v):
    cid = lax.axis_index("c")
    sid = lax.axis_index("s")
    accs = [acc0, acc1, acc2]
    _fill_1d(ones_v, WW, 1.0)
    _fill_1d(zero_v, STRIPE, 0.0)
    for h in range(3):
        pltpu.sync_copy(zero_v, accs[h].at[pl.ds(sid * STRIPE, STRIPE)])
    plsc.subcore_barrier()
    r0 = sid * (EP // 16)
    for h in range(3):
        acc = accs[h]
        for core in range(2):
            t = core * 3 + h
            arr = srcs if t in (0, 2, 4) else dsts
            e = t // 2

            @pl.when(cid == core)
            def _(arr=arr, e=e, acc=acc):
                def body(b, c):
                    pltpu.sync_copy(arr.at[e, pl.ds(r0 + b * WW, WW)], idx_v)
                    pltpu.sync_copy(ones_v, acc.at[idx_v], add=True)
                    return c

                lax.fori_loop(0, EP // 16 // WW, body, 0)

    plsc.subcore_barrier()
    for h in range(3):
        for core in range(2):
            t = core * 3 + h
            arr_is_src = t in (0, 2, 4)
            e = t // 2
            out_row = 2 * e + (0 if arr_is_src else 1)

            @pl.when(cid == core)
            def _(out_row=out_row, h=h):
                pltpu.sync_copy(
                    accs[h].at[pl.ds(sid * STRIPE, STRIPE)],
                    deg_out.at[out_row, 0, pl.ds(sid * STRIPE, STRIPE)],
                )


@functools.partial(
    pl.kernel,
    out_type=jax.ShapeDtypeStruct((NE, NP, 128), jnp.float32),
    mesh=_mesh,
    compiler_params=pltpu.CompilerParams(use_tc_tiling_on_sc=False),
    scratch_types=dict(
        acc=pltpu.VMEM_SHARED((NP, 32), jnp.float32),
        sidxb=pltpu.VMEM((20, 128), jnp.int32),
        didxb=pltpu.VMEM((20, 128), jnp.int32),
        rows0=pltpu.VMEM((128, 32), jnp.float32),
        rows1=pltpu.VMEM((128, 32), jnp.float32),
        zero_v=pltpu.VMEM((98, 32), jnp.float32),
        g0=pltpu.SemaphoreType.DMA,
        g1=pltpu.SemaphoreType.DMA,
        s0=pltpu.SemaphoreType.DMA,
        s1=pltpu.SemaphoreType.DMA,
    ),
)
def _agg1_kernel(y, srcs, dsts, out, acc, sidxb, didxb, rows0, rows1, zero_v,
                 g0, g1, s0, s1):
    cid = lax.axis_index("c")
    sid = lax.axis_index("s")
    _fill_2d(zero_v, 98, 32, 0.0)
    r0 = sid * 80
    for e in range(NE):
        for half in range(2):
            for core in range(2):
                c = core * 2 + half

                @pl.when(cid == core)
                def _(e=e, c=c):
                    for r in range(STRIPE // 98):
                        pltpu.sync_copy(
                            zero_v,
                            acc.at[pl.ds(sid * STRIPE + r * 98, 98)],
                        )
                    plsc.subcore_barrier()

                    for q in range(4):
                        pltpu.sync_copy(
                            srcs.at[c, e, pl.ds(r0 + q * 20, 20)], sidxb)
                        pltpu.sync_copy(
                            dsts.at[e, pl.ds(r0 + q * 20, 20)], didxb)
                        _gs_pipeline(y.at[e], sidxb, didxb, rows0, rows1,
                                     acc, g0, g1, s0, s1, 20)
                    plsc.subcore_barrier()
                    pltpu.sync_copy(
                        acc.at[pl.ds(sid * STRIPE, STRIPE)],
                        out.at[e, pl.ds(sid * STRIPE, STRIPE),
                               pl.ds(c * 32, 32)],
                    )
                    plsc.subcore_barrier()


@functools.partial(
    pl.kernel,
    out_type=jax.ShapeDtypeStruct((2, NP, 128), jnp.float32),
    mesh=_mesh,
    compiler_params=pltpu.CompilerParams(use_tc_tiling_on_sc=False),
    scratch_types=dict(
        acc=pltpu.VMEM_SHARED((NP, 16), jnp.float32),
        sidxb=pltpu.VMEM((40, 128), jnp.int32),
        didxb=pltpu.VMEM((40, 128), jnp.int32),
        rows0=pltpu.VMEM((128, 16), jnp.float32),
        rows1=pltpu.VMEM((128, 16), jnp.float32),
        zero_v=pltpu.VMEM((196, 16), jnp.float32),
        g0=pltpu.SemaphoreType.DMA,
        g1=pltpu.SemaphoreType.DMA,
        s0=pltpu.SemaphoreType.DMA,
        s1=pltpu.SemaphoreType.DMA,
    ),
)
def _agg2_kernel(g, srcs, dsts, out, acc, sidxb, didxb, rows0, rows1, zero_v,
                 g0, g1, s0, s1):
    cid = lax.axis_index("c")
    sid = lax.axis_index("s")
    _fill_2d(zero_v, 196, 16, 0.0)
    r0 = cid * (ER // 2) + sid * 40
    for e in range(NE):
        for r in range(STRIPE // 196):
            pltpu.sync_copy(
                zero_v, acc.at[pl.ds(sid * STRIPE + r * 196, 196)]
            )
        plsc.subcore_barrier()
        pltpu.sync_copy(srcs.at[e, pl.ds(r0, 40)], sidxb)
        pltpu.sync_copy(dsts.at[e, pl.ds(r0, 40)], didxb)
        _gs_pipeline(g, sidxb, didxb, rows0, rows1, acc,
                     g0, g1, s0, s1, 40)
        plsc.subcore_barrier()
        pltpu.sync_copy(
            acc.at[pl.ds(sid * STRIPE, STRIPE)],
            out.at[cid, pl.ds(sid * STRIPE, STRIPE),
                   pl.ds(e * D_OUT, D_OUT)],
        )
        plsc.subcore_barrier()


def _mm1_body(x_ref, w_ref, deg_ref, out_ref):
    y = jnp.dot(x_ref[...], w_ref[...], preferred_element_type=jnp.float32)
    for e in range(NE):
        s = lax.rsqrt(jnp.maximum(deg_ref[2 * e, 0], 1.0))
        out_ref[e] = y[:, e * D_H:(e + 1) * D_H] * s[:, None]


def _mm1(x, w1cat, deg):
    return pl.pallas_call(
        _mm1_body,
        grid=(NBLK,),
        in_specs=[
            pl.BlockSpec((BN, D_IN), lambda i: (i, 0)),
            pl.BlockSpec((D_IN, NE * D_H), lambda i: (0, 0)),
            pl.BlockSpec((6, 1, BN), lambda i: (0, 0, i)),
        ],
        out_specs=pl.BlockSpec((NE, BN, 128), lambda i: (0, i, 0)),
        out_shape=jax.ShapeDtypeStruct((NE, NP, 128), jnp.float32),
    )(x, w1cat, deg)


def _mid_body(agg_ref, deg_ref, b1_ref, w2_ref, g_ref):
    i = pl.program_id(0)
    b1sum = b1_ref[0] + b1_ref[1] + b1_ref[2]
    row = i * BN + lax.broadcasted_iota(jnp.int32, (BN, 1), 0)
    h = jnp.zeros((BN, D_H), jnp.float32)
    for e in range(NE):
        s_in = lax.rsqrt(jnp.maximum(deg_ref[2 * e + 1, 0], 1.0))
        h = h + agg_ref[e] * s_in[:, None]
    h = jnp.maximum(h + b1sum[None, :], 0.0)
    h = jnp.where(row < N, h, 0.0)
    gs = []
    for e in range(NE):
        s_out = lax.rsqrt(jnp.maximum(deg_ref[2 * e, 0], 1.0))
        g = jnp.dot(h, w2_ref[e], preferred_element_type=jnp.float32)
        gs.append(g * s_out[:, None])
    gs.append(jnp.zeros((BN, 128 - NE * D_OUT), jnp.float32))
    g_ref[...] = jnp.concatenate(gs, axis=1)


def _mid(agg, deg, b1s, w2s):
    return pl.pallas_call(
        _mid_body,
        grid=(NBLK,),
        in_specs=[
            pl.BlockSpec((NE, BN, 128), lambda i: (0, i, 0)),
            pl.BlockSpec((6, 1, BN), lambda i: (0, 0, i)),
            pl.BlockSpec((NE, D_H), lambda i: (0, 0)),
            pl.BlockSpec((NE, D_H, D_OUT), lambda i: (0, 0, 0)),
        ],
        out_specs=pl.BlockSpec((BN, 128), lambda i: (i, 0)),
        out_shape=jax.ShapeDtypeStruct((NP, 128), jnp.float32),
    )(agg, deg, b1s, w2s)


def _fin_body(part_ref, deg_ref, b2_ref, out_ref):
    b2sum = b2_ref[0] + b2_ref[1] + b2_ref[2]
    p = part_ref[0] + part_ref[1]
    o = jnp.zeros((BN, D_OUT), jnp.float32)
    for e in range(NE):
        s_in = lax.rsqrt(jnp.maximum(deg_ref[2 * e + 1, 0], 1.0))
        o = o + p[:, e * D_OUT:(e + 1) * D_OUT] * s_in[:, None]
    out_ref[...] = o + b2sum[None, :]


def _fin(part, deg, b2s):
    return pl.pallas_call(
        _fin_body,
        grid=(NBLK,),
        in_specs=[
            pl.BlockSpec((2, BN, 128), lambda i: (0, i, 0)),
            pl.BlockSpec((6, 1, BN), lambda i: (0, 0, i)),
            pl.BlockSpec((NE, D_OUT), lambda i: (0, 0)),
        ],
        out_specs=pl.BlockSpec((BN, D_OUT), lambda i: (i, 0)),
        out_shape=jax.ShapeDtypeStruct((NP, D_OUT), jnp.float32),
    )(part, deg, b2s)


@jax.jit
def kernel(x, edge_index_0, edge_index_1, edge_index_2,
           W1_0, W1_1, W1_2, b1_0, b1_1, b1_2,
           W2_0, W2_1, W2_2, b2_0, b2_1, b2_2):
    pad = N + (jnp.arange(EPAD, dtype=jnp.int32) % PADR)
    srcs, dsts = [], []
    for ei in (edge_index_0, edge_index_1, edge_index_2):
        e32 = ei.astype(jnp.int32)
        srcs.append(jnp.concatenate([e32[0], pad]))
        dsts.append(jnp.concatenate([e32[1], pad]))
    srcs = jnp.stack(srcs)
    dsts = jnp.stack(dsts)
    srcs2d = srcs.reshape(NE, ER, 128)
    dsts2d = dsts.reshape(NE, ER, 128)
    srcs4 = jnp.stack([srcs2d * 4 + c for c in range(4)])
    srcs8 = jnp.stack([srcs2d[e] * 8 + e for e in range(NE)])
    w1cat = jnp.concatenate([W1_0, W1_1, W1_2], axis=1)
    b1s = jnp.stack([b1_0, b1_1, b1_2])
    w2s = jnp.stack([W2_0, W2_1, W2_2])
    b2s = jnp.stack([b2_0, b2_1, b2_2])

    deg3 = _deg_kernel(srcs, dsts)
    y = _mm1(x, w1cat, deg3)
    y4 = y.reshape(NE, NP * 4, 32)
    agg = _agg1_kernel(y4, srcs4, dsts2d)
    g = _mid(agg, deg3, b1s, w2s)
    g16 = g.reshape(NP * 8, D_OUT)
    part = _agg2_kernel(g16, srcs8, dsts2d)
    out = _fin(part, deg3, b2s)
    return out[:N]


# speedup vs baseline: 4.9845x; 1.1219x over previous
import functools
import jax
import jax.numpy as jnp
from jax import lax
from jax.experimental import pallas as pl
from jax.experimental.pallas import tpu as pltpu
from jax.experimental.pallas import tpu_sc as plsc

N = 50000
E = 160000
D_IN = 128
D_H = 128
D_OUT = 16
NE = 3

NP = 50176
STRIPE = NP // 16
ER = 1280
EPAD = ER * 128 - E
PADR = NP - N
EP = ER * 128
ER2 = EP // 256
WW = 512
BN = 512
NBLK = NP // BN

_mesh = plsc.VectorSubcoreMesh(core_axis_name="c", subcore_axis_name="s")


def _fill_1d(ref, n, val):
    v = jnp.full((16,), val, dtype=jnp.float32)

    def body(i, c):
        ref[pl.ds(i * 16, 16)] = v
        return c

    lax.fori_loop(0, n // 16, body, 0)


def _fill_2d(ref, rows, cols, val):
    v = jnp.full((16,), val, dtype=jnp.float32)

    def body(i, c):
        for j in range(cols // 16):
            ref[i, pl.ds(j * 16, 16)] = v
        return c

    lax.fori_loop(0, rows, body, 0)


def _gs_pipeline(src_tab, sidxb, didxb, rows0, rows1, acc, g0, g1, s0, s1, nk):
    nb = nk

    def six(k):
        return sidxb.at[k]

    def dix(k):
        return didxb.at[k]

    pltpu.async_copy(src_tab.at[six(0)], rows0, g0)

    def body(b2, ca):
        k0 = 2 * b2
        k1 = k0 + 1
        pltpu.make_async_copy(src_tab.at[six(k0)], rows0, g0).wait()

        @pl.when(b2 > 0)
        def _():
            pltpu.make_async_copy(rows1, acc.at[dix(k1)], s1).wait()

        pltpu.async_copy(src_tab.at[six(k1)], rows1, g1)
        pltpu.async_copy(rows0, acc.at[dix(k0)], s0, add=True)

        @pl.when(b2 < nb // 2 - 1)
        def _():
            pltpu.make_async_copy(rows0, acc.at[dix(k0)], s0).wait()
            pltpu.async_copy(src_tab.at[six(k0 + 2)], rows0, g0)

        pltpu.make_async_copy(src_tab.at[six(k1)], rows1, g1).wait()
        pltpu.async_copy(rows1, acc.at[dix(k1)], s1, add=True)
        return ca

    lax.fori_loop(0, nb // 2, body, 0)
    pltpu.make_async_copy(rows0, acc.at[dix(0)], s0).wait()
    pltpu.make_async_copy(rows1, acc.at[dix(0)], s1).wait()


@functools.partial(
    pl.kernel,
    out_type=jax.ShapeDtypeStruct((6, 1, NP), jnp.float32),
    mesh=_mesh,
    compiler_params=pltpu.CompilerParams(use_tc_tiling_on_sc=False),
    scratch_types=dict(
        acc0=pltpu.VMEM_SHARED((NP,), jnp.float32),
        acc1=pltpu.VMEM_SHARED((NP,), jnp.float32),
        acc2=pltpu.VMEM_SHARED((NP,), jnp.float32),
        idx_v=pltpu.VMEM((WW,), jnp.int32),
        ones_v=pltpu.VMEM((WW,), jnp.float32),
        zero_v=pltpu.VMEM((STRIPE,), jnp.float32),
    ),
)
def _deg_kernel(srcs, dsts, deg_out, acc0, acc1, acc2, idx_v, ones_v, zero_v):
    cid = lax.axis_index("c")
    sid = lax.axis_index("s")
    accs = [acc0, acc1, acc2]
    _fill_1d(ones_v, WW, 1.0)
    _fill_1d(zero_v, STRIPE, 0.0)
    for h in range(3):
        pltpu.sync_copy(zero_v, accs[h].at[pl.ds(sid * STRIPE, STRIPE)])
    plsc.subcore_barrier()
    r0 = sid * (EP // 16)
    for h in range(3):
        acc = accs[h]
        for core in range(2):
            t = core * 3 + h
            arr = srcs if t in (0, 2, 4) else dsts
            e = t // 2

            @pl.when(cid == core)
            def _(arr=arr, e=e, acc=acc):
                def body(b, c):
                    pltpu.sync_copy(arr.at[e, pl.ds(r0 + b * WW, WW)], idx_v)
                    pltpu.sync_copy(ones_v, acc.at[idx_v], add=True)
                    return c

                lax.fori_loop(0, EP // 16 // WW, body, 0)

    plsc.subcore_barrier()
    for h in range(3):
        for core in range(2):
            t = core * 3 + h
            arr_is_src = t in (0, 2, 4)
            e = t // 2
            out_row = 2 * e + (0 if arr_is_src else 1)

            @pl.when(cid == core)
            def _(out_row=out_row, h=h):
                pltpu.sync_copy(
                    accs[h].at[pl.ds(sid * STRIPE, STRIPE)],
                    deg_out.at[out_row, 0, pl.ds(sid * STRIPE, STRIPE)],
                )


@functools.partial(
    pl.kernel,
    out_type=jax.ShapeDtypeStruct((NE, NP, 128), jnp.float32),
    mesh=_mesh,
    compiler_params=pltpu.CompilerParams(use_tc_tiling_on_sc=False),
    scratch_types=dict(
        acc=pltpu.VMEM_SHARED((NP, 32), jnp.float32),
        sidxb=pltpu.VMEM((10, 256), jnp.int32),
        didxb=pltpu.VMEM((10, 256), jnp.int32),
        rows0=pltpu.VMEM((256, 32), jnp.float32),
        rows1=pltpu.VMEM((256, 32), jnp.float32),
        zero_v=pltpu.VMEM((98, 32), jnp.float32),
        g0=pltpu.SemaphoreType.DMA,
        g1=pltpu.SemaphoreType.DMA,
        s0=pltpu.SemaphoreType.DMA,
        s1=pltpu.SemaphoreType.DMA,
    ),
)
def _agg1_kernel(y, srcs, dsts, out, acc, sidxb, didxb, rows0, rows1, zero_v,
                 g0, g1, s0, s1):
    cid = lax.axis_index("c")
    sid = lax.axis_index("s")
    _fill_2d(zero_v, 98, 32, 0.0)
    r0 = sid * 40
    for e in range(NE):
        for half in range(2):
            for core in range(2):
                c = core * 2 + half

                @pl.when(cid == core)
                def _(e=e, c=c):
                    for r in range(STRIPE // 98):
                        pltpu.sync_copy(
                            zero_v,
                            acc.at[pl.ds(sid * STRIPE + r * 98, 98)],
                        )
                    plsc.subcore_barrier()

                    for q in range(4):
                        pltpu.sync_copy(
                            srcs.at[c, e, pl.ds(r0 + q * 10, 10)], sidxb)
                        pltpu.sync_copy(
                            dsts.at[e, pl.ds(r0 + q * 10, 10)], didxb)
                        _gs_pipeline(y.at[e], sidxb, didxb, rows0, rows1,
                                     acc, g0, g1, s0, s1, 10)
                    plsc.subcore_barrier()
                    pltpu.sync_copy(
                        acc.at[pl.ds(sid * STRIPE, STRIPE)],
                        out.at[e, pl.ds(sid * STRIPE, STRIPE),
                               pl.ds(c * 32, 32)],
                    )
                    plsc.subcore_barrier()


@functools.partial(
    pl.kernel,
    out_type=jax.ShapeDtypeStruct((2, NP, 128), jnp.float32),
    mesh=_mesh,
    compiler_params=pltpu.CompilerParams(use_tc_tiling_on_sc=False),
    scratch_types=dict(
        acc=pltpu.VMEM_SHARED((NP, 16), jnp.float32),
        sidxb=pltpu.VMEM((20, 256), jnp.int32),
        didxb=pltpu.VMEM((20, 256), jnp.int32),
        rows0=pltpu.VMEM((256, 16), jnp.float32),
        rows1=pltpu.VMEM((256, 16), jnp.float32),
        zero_v=pltpu.VMEM((196, 16), jnp.float32),
        g0=pltpu.SemaphoreType.DMA,
        g1=pltpu.SemaphoreType.DMA,
        s0=pltpu.SemaphoreType.DMA,
        s1=pltpu.SemaphoreType.DMA,
    ),
)
def _agg2_kernel(g, srcs, dsts, out, acc, sidxb, didxb, rows0, rows1, zero_v,
                 g0, g1, s0, s1):
    cid = lax.axis_index("c")
    sid = lax.axis_index("s")
    _fill_2d(zero_v, 196, 16, 0.0)
    r0 = cid * (ER2 // 2) + sid * 20
    for e in range(NE):
        for r in range(STRIPE // 196):
            pltpu.sync_copy(
                zero_v, acc.at[pl.ds(sid * STRIPE + r * 196, 196)]
            )
        plsc.subcore_barrier()
        pltpu.sync_copy(srcs.at[e, pl.ds(r0, 20)], sidxb)
        pltpu.sync_copy(dsts.at[e, pl.ds(r0, 20)], didxb)
        _gs_pipeline(g, sidxb, didxb, rows0, rows1, acc,
                     g0, g1, s0, s1, 20)
        plsc.subcore_barrier()
        pltpu.sync_copy(
            acc.at[pl.ds(sid * STRIPE, STRIPE)],
            out.at[cid, pl.ds(sid * STRIPE, STRIPE),
                   pl.ds(e * D_OUT, D_OUT)],
        )
        plsc.subcore_barrier()


def _mm1_body(x_ref, w_ref, deg_ref, out_ref):
    y = jnp.dot(x_ref[...], w_ref[...], preferred_element_type=jnp.float32)
    for e in range(NE):
        s = lax.rsqrt(jnp.maximum(deg_ref[2 * e, 0], 1.0))
        out_ref[e] = y[:, e * D_H:(e + 1) * D_H] * s[:, None]


def _mm1(x, w1cat, deg):
    return pl.pallas_call(
        _mm1_body,
        grid=(NBLK,),
        in_specs=[
            pl.BlockSpec((BN, D_IN), lambda i: (i, 0)),
            pl.BlockSpec((D_IN, NE * D_H), lambda i: (0, 0)),
            pl.BlockSpec((6, 1, BN), lambda i: (0, 0, i)),
        ],
        out_specs=pl.BlockSpec((NE, BN, 128), lambda i: (0, i, 0)),
        out_shape=jax.ShapeDtypeStruct((NE, NP, 128), jnp.float32),
    )(x, w1cat, deg)


def _mid_body(agg_ref, deg_ref, b1_ref, w2_ref, g_ref):
    i = pl.program_id(0)
    b1sum = b1_ref[0] + b1_ref[1] + b1_ref[2]
    row = i * BN + lax.broadcasted_iota(jnp.int32, (BN, 1), 0)
    h = jnp.zeros((BN, D_H), jnp.float32)
    for e in range(NE):
        s_in = lax.rsqrt(jnp.maximum(deg_ref[2 * e + 1, 0], 1.0))
        h = h + agg_ref[e] * s_in[:, None]
    h = jnp.maximum(h + b1sum[None, :], 0.0)
    h = jnp.where(row < N, h, 0.0)
    gs = []
    for e in range(NE):
        s_out = lax.rsqrt(jnp.maximum(deg_ref[2 * e, 0], 1.0))
        g = jnp.dot(h, w2_ref[e], preferred_element_type=jnp.float32)
        gs.append(g * s_out[:, None])
    gs.append(jnp.zeros((BN, 128 - NE * D_OUT), jnp.float32))
    g_ref[...] = jnp.concatenate(gs, axis=1)


def _mid(agg, deg, b1s, w2s):
    return pl.pallas_call(
        _mid_body,
        grid=(NBLK,),
        in_specs=[
            pl.BlockSpec((NE, BN, 128), lambda i: (0, i, 0)),
            pl.BlockSpec((6, 1, BN), lambda i: (0, 0, i)),
            pl.BlockSpec((NE, D_H), lambda i: (0, 0)),
            pl.BlockSpec((NE, D_H, D_OUT), lambda i: (0, 0, 0)),
        ],
        out_specs=pl.BlockSpec((BN, 128), lambda i: (i, 0)),
        out_shape=jax.ShapeDtypeStruct((NP, 128), jnp.float32),
    )(agg, deg, b1s, w2s)


def _fin_body(part_ref, deg_ref, b2_ref, out_ref):
    b2sum = b2_ref[0] + b2_ref[1] + b2_ref[2]
    p = part_ref[0] + part_ref[1]
    o = jnp.zeros((BN, D_OUT), jnp.float32)
    for e in range(NE):
        s_in = lax.rsqrt(jnp.maximum(deg_ref[2 * e + 1, 0], 1.0))
        o = o + p[:, e * D_OUT:(e + 1) * D_OUT] * s_in[:, None]
    out_ref[...] = o + b2sum[None, :]


def _fin(part, deg, b2s):
    return pl.pallas_call(
        _fin_body,
        grid=(NBLK,),
        in_specs=[
            pl.BlockSpec((2, BN, 128), lambda i: (0, i, 0)),
            pl.BlockSpec((6, 1, BN), lambda i: (0, 0, i)),
            pl.BlockSpec((NE, D_OUT), lambda i: (0, 0)),
        ],
        out_specs=pl.BlockSpec((BN, D_OUT), lambda i: (i, 0)),
        out_shape=jax.ShapeDtypeStruct((N, D_OUT), jnp.float32),
    )(part, deg, b2s)


@jax.jit
def kernel(x, edge_index_0, edge_index_1, edge_index_2,
           W1_0, W1_1, W1_2, b1_0, b1_1, b1_2,
           W2_0, W2_1, W2_2, b2_0, b2_1, b2_2):
    pad = N + (jnp.arange(EPAD, dtype=jnp.int32) % PADR)
    srcs, dsts = [], []
    for ei in (edge_index_0, edge_index_1, edge_index_2):
        e32 = ei.astype(jnp.int32)
        srcs.append(jnp.concatenate([e32[0], pad]))
        dsts.append(jnp.concatenate([e32[1], pad]))
    srcs = jnp.stack(srcs)
    dsts = jnp.stack(dsts)
    srcs2d = srcs.reshape(NE, ER2, 256)
    dsts2d = dsts.reshape(NE, ER2, 256)
    srcs4 = jnp.stack([srcs2d * 4 + c for c in range(4)])
    srcs8 = jnp.stack([srcs2d[e] * 8 + e for e in range(NE)])
    w1cat = jnp.concatenate([W1_0, W1_1, W1_2], axis=1)
    b1s = jnp.stack([b1_0, b1_1, b1_2])
    w2s = jnp.stack([W2_0, W2_1, W2_2])
    b2s = jnp.stack([b2_0, b2_1, b2_2])

    deg3 = _deg_kernel(srcs, dsts)
    y = _mm1(x, w1cat, deg3)
    y4 = y.reshape(NE, NP * 4, 32)
    agg = _agg1_kernel(y4, srcs4, dsts2d)
    g = _mid(agg, deg3, b1s, w2s)
    g16 = g.reshape(NP * 8, D_OUT)
    part = _agg2_kernel(g16, srcs8, dsts2d)
    out = _fin(part, deg3, b2s)
    return out


# speedup vs baseline: 5.4546x; 1.0943x over previous
import functools
import jax
import jax.numpy as jnp
from jax import lax
from jax.experimental import pallas as pl
from jax.experimental.pallas import tpu as pltpu
from jax.experimental.pallas import tpu_sc as plsc

N = 50000
E = 160000
D_IN = 128
D_H = 128
D_OUT = 16
NE = 3

NP = 50176
STRIPE = NP // 16
ER = 1280
EPAD = ER * 128 - E
PADR = NP - N
EP = ER * 128
ER2 = EP // 256
WW = 512
BN = 1024
NBLK = NP // BN

_mesh = plsc.VectorSubcoreMesh(core_axis_name="c", subcore_axis_name="s")


def _fill_1d(ref, n, val):
    v = jnp.full((16,), val, dtype=jnp.float32)

    def body(i, c):
        ref[pl.ds(i * 16, 16)] = v
        return c

    lax.fori_loop(0, n // 16, body, 0)


def _fill_2d(ref, rows, cols, val):
    v = jnp.full((16,), val, dtype=jnp.float32)

    def body(i, c):
        for j in range(cols // 16):
            ref[i, pl.ds(j * 16, 16)] = v
        return c

    lax.fori_loop(0, rows, body, 0)


def _gs_pipeline(src_tab, sidxb, didxb, rows0, rows1, acc, g0, g1, s0, s1, nk):
    nb = nk

    def six(k):
        return sidxb.at[k]

    def dix(k):
        return didxb.at[k]

    pltpu.async_copy(src_tab.at[six(0)], rows0, g0)

    def body(b2, ca):
        k0 = 2 * b2
        k1 = k0 + 1
        pltpu.make_async_copy(src_tab.at[six(k0)], rows0, g0).wait()

        @pl.when(b2 > 0)
        def _():
            pltpu.make_async_copy(rows1, acc.at[dix(k1)], s1).wait()

        pltpu.async_copy(src_tab.at[six(k1)], rows1, g1)
        pltpu.async_copy(rows0, acc.at[dix(k0)], s0, add=True)

        @pl.when(b2 < nb // 2 - 1)
        def _():
            pltpu.make_async_copy(rows0, acc.at[dix(k0)], s0).wait()
            pltpu.async_copy(src_tab.at[six(k0 + 2)], rows0, g0)

        pltpu.make_async_copy(src_tab.at[six(k1)], rows1, g1).wait()
        pltpu.async_copy(rows1, acc.at[dix(k1)], s1, add=True)
        return ca

    lax.fori_loop(0, nb // 2, body, 0)
    pltpu.make_async_copy(rows0, acc.at[dix(0)], s0).wait()
    pltpu.make_async_copy(rows1, acc.at[dix(0)], s1).wait()


@functools.partial(
    pl.kernel,
    out_type=jax.ShapeDtypeStruct((6, 1, NP), jnp.float32),
    mesh=_mesh,
    compiler_params=pltpu.CompilerParams(use_tc_tiling_on_sc=False),
    scratch_types=dict(
        acc0=pltpu.VMEM_SHARED((NP,), jnp.float32),
        acc1=pltpu.VMEM_SHARED((NP,), jnp.float32),
        acc2=pltpu.VMEM_SHARED((NP,), jnp.float32),
        idx_v=pltpu.VMEM((WW,), jnp.int32),
        ones_v=pltpu.VMEM((WW,), jnp.float32),
        zero_v=pltpu.VMEM((STRIPE,), jnp.float32),
    ),
)
def _deg_kernel(srcs, dsts, deg_out, acc0, acc1, acc2, idx_v, ones_v, zero_v):
    cid = lax.axis_index("c")
    sid = lax.axis_index("s")
    accs = [acc0, acc1, acc2]
    _fill_1d(ones_v, WW, 1.0)
    _fill_1d(zero_v, STRIPE, 0.0)
    for h in range(3):
        pltpu.sync_copy(zero_v, accs[h].at[pl.ds(sid * STRIPE, STRIPE)])
    plsc.subcore_barrier()
    r0 = sid * (EP // 16)
    for h in range(3):
        acc = accs[h]
        for core in range(2):
            t = core * 3 + h
            arr = srcs if t in (0, 2, 4) else dsts
            e = t // 2

            @pl.when(cid == core)
            def _(arr=arr, e=e, acc=acc):
                def body(b, c):
                    pltpu.sync_copy(arr.at[e, pl.ds(r0 + b * WW, WW)], idx_v)
                    pltpu.sync_copy(ones_v, acc.at[idx_v], add=True)
                    return c

                lax.fori_loop(0, EP // 16 // WW, body, 0)

    plsc.subcore_barrier()
    for h in range(3):
        for core in range(2):
            t = core * 3 + h
            arr_is_src = t in (0, 2, 4)
            e = t // 2
            out_row = 2 * e + (0 if arr_is_src else 1)

            @pl.when(cid == core)
            def _(out_row=out_row, h=h):
                pltpu.sync_copy(
                    accs[h].at[pl.ds(sid * STRIPE, STRIPE)],
                    deg_out.at[out_row, 0, pl.ds(sid * STRIPE, STRIPE)],
                )


@functools.partial(
    pl.kernel,
    out_type=jax.ShapeDtypeStruct((NE, NP, 128), jnp.float32),
    mesh=_mesh,
    compiler_params=pltpu.CompilerParams(use_tc_tiling_on_sc=False),
    scratch_types=dict(
        acc=pltpu.VMEM_SHARED((NP, 32), jnp.float32),
        sidxb=pltpu.VMEM((10, 256), jnp.int32),
        didxb=pltpu.VMEM((10, 256), jnp.int32),
        rows0=pltpu.VMEM((256, 32), jnp.float32),
        rows1=pltpu.VMEM((256, 32), jnp.float32),
        zero_v=pltpu.VMEM((98, 32), jnp.float32),
        g0=pltpu.SemaphoreType.DMA,
        g1=pltpu.SemaphoreType.DMA,
        s0=pltpu.SemaphoreType.DMA,
        s1=pltpu.SemaphoreType.DMA,
    ),
)
def _agg1_kernel(y, srcs, dsts, out, acc, sidxb, didxb, rows0, rows1, zero_v,
                 g0, g1, s0, s1):
    cid = lax.axis_index("c")
    sid = lax.axis_index("s")
    _fill_2d(zero_v, 98, 32, 0.0)
    r0 = sid * 40
    for e in range(NE):
        for half in range(2):
            for core in range(2):
                c = core * 2 + half

                @pl.when(cid == core)
                def _(e=e, c=c):
                    for r in range(STRIPE // 98):
                        pltpu.sync_copy(
                            zero_v,
                            acc.at[pl.ds(sid * STRIPE + r * 98, 98)],
                        )
                    plsc.subcore_barrier()

                    for q in range(4):
                        pltpu.sync_copy(
                            srcs.at[c, e, pl.ds(r0 + q * 10, 10)], sidxb)
                        pltpu.sync_copy(
                            dsts.at[e, pl.ds(r0 + q * 10, 10)], didxb)
                        _gs_pipeline(y.at[e], sidxb, didxb, rows0, rows1,
                                     acc, g0, g1, s0, s1, 10)
                    plsc.subcore_barrier()
                    pltpu.sync_copy(
                        acc.at[pl.ds(sid * STRIPE, STRIPE)],
                        out.at[e, pl.ds(sid * STRIPE, STRIPE),
                               pl.ds(c * 32, 32)],
                    )
                    plsc.subcore_barrier()


@functools.partial(
    pl.kernel,
    out_type=jax.ShapeDtypeStruct((2, NP, 128), jnp.float32),
    mesh=_mesh,
    compiler_params=pltpu.CompilerParams(use_tc_tiling_on_sc=False),
    scratch_types=dict(
        acc=pltpu.VMEM_SHARED((NP, 16), jnp.float32),
        sidxb=pltpu.VMEM((20, 256), jnp.int32),
        didxb=pltpu.VMEM((20, 256), jnp.int32),
        rows0=pltpu.VMEM((256, 16), jnp.float32),
        rows1=pltpu.VMEM((256, 16), jnp.float32),
        zero_v=pltpu.VMEM((196, 16), jnp.float32),
        g0=pltpu.SemaphoreType.DMA,
        g1=pltpu.SemaphoreType.DMA,
        s0=pltpu.SemaphoreType.DMA,
        s1=pltpu.SemaphoreType.DMA,
    ),
)
def _agg2_kernel(g, srcs, dsts, out, acc, sidxb, didxb, rows0, rows1, zero_v,
                 g0, g1, s0, s1):
    cid = lax.axis_index("c")
    sid = lax.axis_index("s")
    _fill_2d(zero_v, 196, 16, 0.0)
    r0 = cid * (ER2 // 2) + sid * 20
    for e in range(NE):
        for r in range(STRIPE // 196):
            pltpu.sync_copy(
                zero_v, acc.at[pl.ds(sid * STRIPE + r * 196, 196)]
            )
        plsc.subcore_barrier()
        pltpu.sync_copy(srcs.at[e, pl.ds(r0, 20)], sidxb)
        pltpu.sync_copy(dsts.at[e, pl.ds(r0, 20)], didxb)
        _gs_pipeline(g, sidxb, didxb, rows0, rows1, acc,
                     g0, g1, s0, s1, 20)
        plsc.subcore_barrier()
        pltpu.sync_copy(
            acc.at[pl.ds(sid * STRIPE, STRIPE)],
            out.at[cid, pl.ds(sid * STRIPE, STRIPE),
                   pl.ds(e * D_OUT, D_OUT)],
        )
        plsc.subcore_barrier()


def _mm1_body(x_ref, w_ref, deg_ref, out_ref):
    y = jnp.dot(x_ref[...], w_ref[...], preferred_element_type=jnp.float32)
    for e in range(NE):
        s = lax.rsqrt(jnp.maximum(deg_ref[2 * e, 0], 1.0))
        out_ref[e] = y[:, e * D_H:(e + 1) * D_H] * s[:, None]


def _mm1(x, w1cat, deg):
    return pl.pallas_call(
        _mm1_body,
        grid=(NBLK,),
        in_specs=[
            pl.BlockSpec((BN, D_IN), lambda i: (i, 0)),
            pl.BlockSpec((D_IN, NE * D_H), lambda i: (0, 0)),
            pl.BlockSpec((6, 1, BN), lambda i: (0, 0, i)),
        ],
        out_specs=pl.BlockSpec((NE, BN, 128), lambda i: (0, i, 0)),
        out_shape=jax.ShapeDtypeStruct((NE, NP, 128), jnp.float32),
    )(x, w1cat, deg)


def _mid_body(agg_ref, deg_ref, b1_ref, w2_ref, g_ref):
    i = pl.program_id(0)
    b1sum = b1_ref[0] + b1_ref[1] + b1_ref[2]
    row = i * BN + lax.broadcasted_iota(jnp.int32, (BN, 1), 0)
    h = jnp.zeros((BN, D_H), jnp.float32)
    for e in range(NE):
        s_in = lax.rsqrt(jnp.maximum(deg_ref[2 * e + 1, 0], 1.0))
        h = h + agg_ref[e] * s_in[:, None]
    h = jnp.maximum(h + b1sum[None, :], 0.0)
    h = jnp.where(row < N, h, 0.0)
    gs = []
    for e in range(NE):
        s_out = lax.rsqrt(jnp.maximum(deg_ref[2 * e, 0], 1.0))
        g = jnp.dot(h, w2_ref[e], preferred_element_type=jnp.float32)
        gs.append(g * s_out[:, None])
    gs.append(jnp.zeros((BN, 128 - NE * D_OUT), jnp.float32))
    g_ref[...] = jnp.concatenate(gs, axis=1)


def _mid(agg, deg, b1s, w2s):
    return pl.pallas_call(
        _mid_body,
        grid=(NBLK,),
        in_specs=[
            pl.BlockSpec((NE, BN, 128), lambda i: (0, i, 0)),
            pl.BlockSpec((6, 1, BN), lambda i: (0, 0, i)),
            pl.BlockSpec((NE, D_H), lambda i: (0, 0)),
            pl.BlockSpec((NE, D_H, D_OUT), lambda i: (0, 0, 0)),
        ],
        out_specs=pl.BlockSpec((BN, 128), lambda i: (i, 0)),
        out_shape=jax.ShapeDtypeStruct((NP, 128), jnp.float32),
    )(agg, deg, b1s, w2s)


def _fin_body(part_ref, deg_ref, b2_ref, out_ref):
    b2sum = b2_ref[0] + b2_ref[1] + b2_ref[2]
    p = part_ref[0] + part_ref[1]
    o = jnp.zeros((BN, D_OUT), jnp.float32)
    for e in range(NE):
        s_in = lax.rsqrt(jnp.maximum(deg_ref[2 * e + 1, 0], 1.0))
        o = o + p[:, e * D_OUT:(e + 1) * D_OUT] * s_in[:, None]
    out_ref[...] = o + b2sum[None, :]


def _fin(part, deg, b2s):
    return pl.pallas_call(
        _fin_body,
        grid=(NBLK,),
        in_specs=[
            pl.BlockSpec((2, BN, 128), lambda i: (0, i, 0)),
            pl.BlockSpec((6, 1, BN), lambda i: (0, 0, i)),
            pl.BlockSpec((NE, D_OUT), lambda i: (0, 0)),
        ],
        out_specs=pl.BlockSpec((BN, D_OUT), lambda i: (i, 0)),
        out_shape=jax.ShapeDtypeStruct((N, D_OUT), jnp.float32),
    )(part, deg, b2s)


@jax.jit
def kernel(x, edge_index_0, edge_index_1, edge_index_2,
           W1_0, W1_1, W1_2, b1_0, b1_1, b1_2,
           W2_0, W2_1, W2_2, b2_0, b2_1, b2_2):
    pad = N + (jnp.arange(EPAD, dtype=jnp.int32) % PADR)
    srcs, dsts = [], []
    for ei in (edge_index_0, edge_index_1, edge_index_2):
        e32 = ei.astype(jnp.int32)
        srcs.append(jnp.concatenate([e32[0], pad]))
        dsts.append(jnp.concatenate([e32[1], pad]))
    srcs = jnp.stack(srcs)
    dsts = jnp.stack(dsts)
    srcs2d = srcs.reshape(NE, ER2, 256)
    dsts2d = dsts.reshape(NE, ER2, 256)
    srcs4 = jnp.stack([srcs2d * 4 + c for c in range(4)])
    srcs8 = jnp.stack([srcs2d[e] * 8 + e for e in range(NE)])
    w1cat = jnp.concatenate([W1_0, W1_1, W1_2], axis=1)
    b1s = jnp.stack([b1_0, b1_1, b1_2])
    w2s = jnp.stack([W2_0, W2_1, W2_2])
    b2s = jnp.stack([b2_0, b2_1, b2_2])

    deg3 = _deg_kernel(srcs, dsts)
    y = _mm1(x, w1cat, deg3)
    y4 = y.reshape(NE, NP * 4, 32)
    agg = _agg1_kernel(y4, srcs4, dsts2d)
    g = _mid(agg, deg3, b1s, w2s)
    g16 = g.reshape(NP * 8, D_OUT)
    part = _agg2_kernel(g16, srcs8, dsts2d)
    out = _fin(part, deg3, b2s)
    return out


# speedup vs baseline: 5.5474x; 1.0170x over previous
import functools
import jax
import jax.numpy as jnp
from jax import lax
from jax.experimental import pallas as pl
from jax.experimental.pallas import tpu as pltpu
from jax.experimental.pallas import tpu_sc as plsc

N = 50000
E = 160000
D_IN = 128
D_H = 128
D_OUT = 16
NE = 3

NP = 50176
STRIPE = NP // 16
ER = 1280
EPAD = ER * 128 - E
PADR = NP - N
EP = ER * 128
ER2 = EP // 256
WW = 512
BN = 1024
NBLK = NP // BN

_mesh = plsc.VectorSubcoreMesh(core_axis_name="c", subcore_axis_name="s")


def _fill_1d(ref, n, val):
    v = jnp.full((16,), val, dtype=jnp.float32)

    def body(i, c):
        ref[pl.ds(i * 16, 16)] = v
        return c

    lax.fori_loop(0, n // 16, body, 0)


def _fill_2d(ref, rows, cols, val):
    v = jnp.full((16,), val, dtype=jnp.float32)

    def body(i, c):
        for j in range(cols // 16):
            ref[i, pl.ds(j * 16, 16)] = v
        return c

    lax.fori_loop(0, rows, body, 0)


def _gs_pipeline(src_tab, sidxb, didxb, rows0, rows1, acc, g0, g1, s0, s1, nk):
    nb = nk

    def six(k):
        return sidxb.at[k]

    def dix(k):
        return didxb.at[k]

    pltpu.async_copy(src_tab.at[six(0)], rows0, g0)

    def body(b2, ca):
        k0 = 2 * b2
        k1 = k0 + 1
        pltpu.make_async_copy(src_tab.at[six(k0)], rows0, g0).wait()

        @pl.when(b2 > 0)
        def _():
            pltpu.make_async_copy(rows1, acc.at[dix(k1)], s1).wait()

        pltpu.async_copy(src_tab.at[six(k1)], rows1, g1)
        pltpu.async_copy(rows0, acc.at[dix(k0)], s0, add=True)

        @pl.when(b2 < nb // 2 - 1)
        def _():
            pltpu.make_async_copy(rows0, acc.at[dix(k0)], s0).wait()
            pltpu.async_copy(src_tab.at[six(k0 + 2)], rows0, g0)

        pltpu.make_async_copy(src_tab.at[six(k1)], rows1, g1).wait()
        pltpu.async_copy(rows1, acc.at[dix(k1)], s1, add=True)
        return ca

    lax.fori_loop(0, nb // 2, body, 0)
    pltpu.make_async_copy(rows0, acc.at[dix(0)], s0).wait()
    pltpu.make_async_copy(rows1, acc.at[dix(0)], s1).wait()


@functools.partial(
    pl.kernel,
    out_type=jax.ShapeDtypeStruct((6, 1, NP), jnp.float32),
    mesh=_mesh,
    compiler_params=pltpu.CompilerParams(use_tc_tiling_on_sc=False),
    scratch_types=dict(
        acc0=pltpu.VMEM_SHARED((NP,), jnp.float32),
        acc1=pltpu.VMEM_SHARED((NP,), jnp.float32),
        acc2=pltpu.VMEM_SHARED((NP,), jnp.float32),
        idx_v=pltpu.VMEM((WW,), jnp.int32),
        ones_v=pltpu.VMEM((WW,), jnp.float32),
        zero_v=pltpu.VMEM((STRIPE,), jnp.float32),
    ),
)
def _deg_kernel(srcs, dsts, deg_out, acc0, acc1, acc2, idx_v, ones_v, zero_v):
    cid = lax.axis_index("c")
    sid = lax.axis_index("s")
    accs = [acc0, acc1, acc2]
    _fill_1d(ones_v, WW, 1.0)
    _fill_1d(zero_v, STRIPE, 0.0)
    for h in range(3):
        pltpu.sync_copy(zero_v, accs[h].at[pl.ds(sid * STRIPE, STRIPE)])
    plsc.subcore_barrier()
    r0 = sid * (EP // 16)
    for h in range(3):
        acc = accs[h]
        for core in range(2):
            t = core * 3 + h
            arr = srcs if t in (0, 2, 4) else dsts
            e = t // 2

            @pl.when(cid == core)
            def _(arr=arr, e=e, acc=acc):
                def body(b, c):
                    pltpu.sync_copy(arr.at[e, pl.ds(r0 + b * WW, WW)], idx_v)
                    pltpu.sync_copy(ones_v, acc.at[idx_v], add=True)
                    return c

                lax.fori_loop(0, EP // 16 // WW, body, 0)

    plsc.subcore_barrier()
    for h in range(3):
        for core in range(2):
            t = core * 3 + h
            arr_is_src = t in (0, 2, 4)
            e = t // 2
            out_row = 2 * e + (0 if arr_is_src else 1)

            @pl.when(cid == core)
            def _(out_row=out_row, h=h):
                pltpu.sync_copy(
                    accs[h].at[pl.ds(sid * STRIPE, STRIPE)],
                    deg_out.at[out_row, 0, pl.ds(sid * STRIPE, STRIPE)],
                )


@functools.partial(
    pl.kernel,
    out_type=jax.ShapeDtypeStruct((NE, NP, 128), jnp.float32),
    mesh=_mesh,
    compiler_params=pltpu.CompilerParams(use_tc_tiling_on_sc=False),
    scratch_types=dict(
        acc=pltpu.VMEM_SHARED((NP, 32), jnp.float32),
        sidxb=pltpu.VMEM((10, 256), jnp.int32),
        didxb=pltpu.VMEM((10, 256), jnp.int32),
        rows0=pltpu.VMEM((256, 32), jnp.float32),
        rows1=pltpu.VMEM((256, 32), jnp.float32),
        zero_v=pltpu.VMEM((98, 32), jnp.float32),
        g0=pltpu.SemaphoreType.DMA,
        g1=pltpu.SemaphoreType.DMA,
        s0=pltpu.SemaphoreType.DMA,
        s1=pltpu.SemaphoreType.DMA,
    ),
)
def _agg1_kernel(y, srcs, dsts, out, acc, sidxb, didxb, rows0, rows1, zero_v,
                 g0, g1, s0, s1):
    cid = lax.axis_index("c")
    sid = lax.axis_index("s")
    _fill_2d(zero_v, 98, 32, 0.0)
    r0 = sid * 40
    for e in range(NE):
        for half in range(2):
            for core in range(2):
                c = core * 2 + half

                @pl.when(cid == core)
                def _(e=e, c=c):
                    for r in range(STRIPE // 98):
                        pltpu.async_copy(
                            zero_v,
                            acc.at[pl.ds(sid * STRIPE + r * 98, 98)], s0,
                        )
                    for r in range(STRIPE // 98):
                        pltpu.make_async_copy(
                            zero_v,
                            acc.at[pl.ds(sid * STRIPE + r * 98, 98)], s0,
                        ).wait()
                    plsc.subcore_barrier()

                    for q in range(4):
                        pltpu.sync_copy(
                            srcs.at[c, e, pl.ds(r0 + q * 10, 10)], sidxb)
                        pltpu.sync_copy(
                            dsts.at[e, pl.ds(r0 + q * 10, 10)], didxb)
                        _gs_pipeline(y.at[e], sidxb, didxb, rows0, rows1,
                                     acc, g0, g1, s0, s1, 10)
                    plsc.subcore_barrier()
                    pltpu.sync_copy(
                        acc.at[pl.ds(sid * STRIPE, STRIPE)],
                        out.at[e, pl.ds(sid * STRIPE, STRIPE),
                               pl.ds(c * 32, 32)],
                    )
                    plsc.subcore_barrier()


@functools.partial(
    pl.kernel,
    out_type=jax.ShapeDtypeStruct((2, NP, 128), jnp.float32),
    mesh=_mesh,
    compiler_params=pltpu.CompilerParams(use_tc_tiling_on_sc=False),
    scratch_types=dict(
        acc=pltpu.VMEM_SHARED((NP, 16), jnp.float32),
        sidxb=pltpu.VMEM((20, 256), jnp.int32),
        didxb=pltpu.VMEM((20, 256), jnp.int32),
        rows0=pltpu.VMEM((256, 16), jnp.float32),
        rows1=pltpu.VMEM((256, 16), jnp.float32),
        zero_v=pltpu.VMEM((196, 16), jnp.float32),
        g0=pltpu.SemaphoreType.DMA,
        g1=pltpu.SemaphoreType.DMA,
        s0=pltpu.SemaphoreType.DMA,
        s1=pltpu.SemaphoreType.DMA,
    ),
)
def _agg2_kernel(g, srcs, dsts, out, acc, sidxb, didxb, rows0, rows1, zero_v,
                 g0, g1, s0, s1):
    cid = lax.axis_index("c")
    sid = lax.axis_index("s")
    _fill_2d(zero_v, 196, 16, 0.0)
    r0 = cid * (ER2 // 2) + sid * 20
    for e in range(NE):
        for r in range(STRIPE // 196):
            pltpu.async_copy(
                zero_v, acc.at[pl.ds(sid * STRIPE + r * 196, 196)], s0
            )
        for r in range(STRIPE // 196):
            pltpu.make_async_copy(
                zero_v, acc.at[pl.ds(sid * STRIPE + r * 196, 196)], s0
            ).wait()
        plsc.subcore_barrier()
        pltpu.sync_copy(srcs.at[e, pl.ds(r0, 20)], sidxb)
        pltpu.sync_copy(dsts.at[e, pl.ds(r0, 20)], didxb)
        _gs_pipeline(g, sidxb, didxb, rows0, rows1, acc,
                     g0, g1, s0, s1, 20)
        plsc.subcore_barrier()
        pltpu.sync_copy(
            acc.at[pl.ds(sid * STRIPE, STRIPE)],
            out.at[cid, pl.ds(sid * STRIPE, STRIPE),
                   pl.ds(e * D_OUT, D_OUT)],
        )
        plsc.subcore_barrier()


def _mm1_body(x_ref, w_ref, deg_ref, out_ref):
    y = jnp.dot(x_ref[...], w_ref[...], preferred_element_type=jnp.float32)
    for e in range(NE):
        s = lax.rsqrt(jnp.maximum(deg_ref[2 * e, 0], 1.0))
        out_ref[e] = y[:, e * D_H:(e + 1) * D_H] * s[:, None]


def _mm1(x, w1cat, deg):
    return pl.pallas_call(
        _mm1_body,
        grid=(NBLK,),
        in_specs=[
            pl.BlockSpec((BN, D_IN), lambda i: (i, 0)),
            pl.BlockSpec((D_IN, NE * D_H), lambda i: (0, 0)),
            pl.BlockSpec((6, 1, BN), lambda i: (0, 0, i)),
        ],
        out_specs=pl.BlockSpec((NE, BN, 128), lambda i: (0, i, 0)),
        out_shape=jax.ShapeDtypeStruct((NE, NP, 128), jnp.float32),
    )(x, w1cat, deg)


def _mid_body(agg_ref, deg_ref, b1_ref, w2_ref, g_ref):
    i = pl.program_id(0)
    b1sum = b1_ref[0] + b1_ref[1] + b1_ref[2]
    row = i * BN + lax.broadcasted_iota(jnp.int32, (BN, 1), 0)
    h = jnp.zeros((BN, D_H), jnp.float32)
    for e in range(NE):
        s_in = lax.rsqrt(jnp.maximum(deg_ref[2 * e + 1, 0], 1.0))
        h = h + agg_ref[e] * s_in[:, None]
    h = jnp.maximum(h + b1sum[None, :], 0.0)
    h = jnp.where(row < N, h, 0.0)
    gs = []
    for e in range(NE):
        s_out = lax.rsqrt(jnp.maximum(deg_ref[2 * e, 0], 1.0))
        g = jnp.dot(h, w2_ref[e], preferred_element_type=jnp.float32)
        gs.append(g * s_out[:, None])
    gs.append(jnp.zeros((BN, 128 - NE * D_OUT), jnp.float32))
    g_ref[...] = jnp.concatenate(gs, axis=1)


def _mid(agg, deg, b1s, w2s):
    return pl.pallas_call(
        _mid_body,
        grid=(NBLK,),
        in_specs=[
            pl.BlockSpec((NE, BN, 128), lambda i: (0, i, 0)),
            pl.BlockSpec((6, 1, BN), lambda i: (0, 0, i)),
            pl.BlockSpec((NE, D_H), lambda i: (0, 0)),
            pl.BlockSpec((NE, D_H, D_OUT), lambda i: (0, 0, 0)),
        ],
        out_specs=pl.BlockSpec((BN, 128), lambda i: (i, 0)),
        out_shape=jax.ShapeDtypeStruct((NP, 128), jnp.float32),
    )(agg, deg, b1s, w2s)


def _fin_body(part_ref, deg_ref, b2_ref, out_ref):
    b2sum = b2_ref[0] + b2_ref[1] + b2_ref[2]
    p = part_ref[0] + part_ref[1]
    o = jnp.zeros((BN, D_OUT), jnp.float32)
    for e in range(NE):
        s_in = lax.rsqrt(jnp.maximum(deg_ref[2 * e + 1, 0], 1.0))
        o = o + p[:, e * D_OUT:(e + 1) * D_OUT] * s_in[:, None]
    out_ref[...] = o + b2sum[None, :]


def _fin(part, deg, b2s):
    return pl.pallas_call(
        _fin_body,
        grid=(NBLK,),
        in_specs=[
            pl.BlockSpec((2, BN, 128), lambda i: (0, i, 0)),
            pl.BlockSpec((6, 1, BN), lambda i: (0, 0, i)),
            pl.BlockSpec((NE, D_OUT), lambda i: (0, 0)),
        ],
        out_specs=pl.BlockSpec((BN, D_OUT), lambda i: (i, 0)),
        out_shape=jax.ShapeDtypeStruct((N, D_OUT), jnp.float32),
    )(part, deg, b2s)


@jax.jit
def kernel(x, edge_index_0, edge_index_1, edge_index_2,
           W1_0, W1_1, W1_2, b1_0, b1_1, b1_2,
           W2_0, W2_1, W2_2, b2_0, b2_1, b2_2):
    pad = N + (jnp.arange(EPAD, dtype=jnp.int32) % PADR)
    srcs, dsts = [], []
    for ei in (edge_index_0, edge_index_1, edge_index_2):
        e32 = ei.astype(jnp.int32)
        srcs.append(jnp.concatenate([e32[0], pad]))
        dsts.append(jnp.concatenate([e32[1], pad]))
    srcs = jnp.stack(srcs)
    dsts = jnp.stack(dsts)
    srcs2d = srcs.reshape(NE, ER2, 256)
    dsts2d = dsts.reshape(NE, ER2, 256)
    srcs4 = jnp.stack([srcs2d * 4 + c for c in range(4)])
    srcs8 = jnp.stack([srcs2d[e] * 8 + e for e in range(NE)])
    w1cat = jnp.concatenate([W1_0, W1_1, W1_2], axis=1)
    b1s = jnp.stack([b1_0, b1_1, b1_2])
    w2s = jnp.stack([W2_0, W2_1, W2_2])
    b2s = jnp.stack([b2_0, b2_1, b2_2])

    deg3 = _deg_kernel(srcs, dsts)
    y = _mm1(x, w1cat, deg3)
    y4 = y.reshape(NE, NP * 4, 32)
    agg = _agg1_kernel(y4, srcs4, dsts2d)
    g = _mid(agg, deg3, b1s, w2s)
    g16 = g.reshape(NP * 8, D_OUT)
    part = _agg2_kernel(g16, srcs8, dsts2d)
    out = _fin(part, deg3, b2s)
    return out


# speedup vs baseline: 5.6849x; 1.0248x over previous
import functools
import jax
import jax.numpy as jnp
from jax import lax
from jax.experimental import pallas as pl
from jax.experimental.pallas import tpu as pltpu
from jax.experimental.pallas import tpu_sc as plsc

N = 50000
E = 160000
D_IN = 128
D_H = 128
D_OUT = 16
NE = 3

NP = 50176
STRIPE = NP // 16
ER = 1280
EPAD = ER * 128 - E
PADR = NP - N
EP = ER * 128
ER2 = EP // 256
WW = 512
BN = 1024
NBLK = NP // BN

_mesh = plsc.VectorSubcoreMesh(core_axis_name="c", subcore_axis_name="s")


def _fill_1d(ref, n, val):
    v = jnp.full((16,), val, dtype=jnp.float32)

    def body(i, c):
        ref[pl.ds(i * 16, 16)] = v
        return c

    lax.fori_loop(0, n // 16, body, 0)


def _fill_2d(ref, rows, cols, val):
    v = jnp.full((16,), val, dtype=jnp.float32)

    def body(i, c):
        for j in range(cols // 16):
            ref[i, pl.ds(j * 16, 16)] = v
        return c

    lax.fori_loop(0, rows, body, 0)


def _gs_pipeline(src_tab, sidxb, didxb, rows0, rows1, acc, g0, g1, s0, s1, nk):
    nb = nk

    def six(k):
        return sidxb.at[k]

    def dix(k):
        return didxb.at[k]

    pltpu.async_copy(src_tab.at[six(0)], rows0, g0)

    def body(b2, ca):
        k0 = 2 * b2
        k1 = k0 + 1
        pltpu.make_async_copy(src_tab.at[six(k0)], rows0, g0).wait()

        @pl.when(b2 > 0)
        def _():
            pltpu.make_async_copy(rows1, acc.at[dix(k1)], s1).wait()

        pltpu.async_copy(src_tab.at[six(k1)], rows1, g1)
        pltpu.async_copy(rows0, acc.at[dix(k0)], s0, add=True)

        @pl.when(b2 < nb // 2 - 1)
        def _():
            pltpu.make_async_copy(rows0, acc.at[dix(k0)], s0).wait()
            pltpu.async_copy(src_tab.at[six(k0 + 2)], rows0, g0)

        pltpu.make_async_copy(src_tab.at[six(k1)], rows1, g1).wait()
        pltpu.async_copy(rows1, acc.at[dix(k1)], s1, add=True)
        return ca

    lax.fori_loop(0, nb // 2, body, 0)
    pltpu.make_async_copy(rows0, acc.at[dix(0)], s0).wait()
    pltpu.make_async_copy(rows1, acc.at[dix(0)], s1).wait()


@functools.partial(
    pl.kernel,
    out_type=jax.ShapeDtypeStruct((6, 1, NP), jnp.float32),
    mesh=_mesh,
    compiler_params=pltpu.CompilerParams(use_tc_tiling_on_sc=False),
    scratch_types=dict(
        acc0=pltpu.VMEM_SHARED((NP,), jnp.float32),
        acc1=pltpu.VMEM_SHARED((NP,), jnp.float32),
        acc2=pltpu.VMEM_SHARED((NP,), jnp.float32),
        idx_v=pltpu.VMEM((WW,), jnp.int32),
        ones_v=pltpu.VMEM((WW,), jnp.float32),
        zero_v=pltpu.VMEM((STRIPE,), jnp.float32),
    ),
)
def _deg_kernel(srcs, dsts, deg_out, acc0, acc1, acc2, idx_v, ones_v, zero_v):
    cid = lax.axis_index("c")
    sid = lax.axis_index("s")
    accs = [acc0, acc1, acc2]
    _fill_1d(ones_v, WW, 1.0)
    _fill_1d(zero_v, STRIPE, 0.0)
    for h in range(3):
        pltpu.sync_copy(zero_v, accs[h].at[pl.ds(sid * STRIPE, STRIPE)])
    plsc.subcore_barrier()
    r0 = sid * (EP // 16)
    for h in range(3):
        acc = accs[h]
        for core in range(2):
            t = core * 3 + h
            arr = srcs if t in (0, 2, 4) else dsts
            e = t // 2

            @pl.when(cid == core)
            def _(arr=arr, e=e, acc=acc):
                def body(b, c):
                    pltpu.sync_copy(arr.at[e, pl.ds(r0 + b * WW, WW)], idx_v)
                    pltpu.sync_copy(ones_v, acc.at[idx_v], add=True)
                    return c

                lax.fori_loop(0, EP // 16 // WW, body, 0)

    plsc.subcore_barrier()
    for h in range(3):
        for core in range(2):
            t = core * 3 + h
            arr_is_src = t in (0, 2, 4)
            e = t // 2
            out_row = 2 * e + (0 if arr_is_src else 1)

            @pl.when(cid == core)
            def _(out_row=out_row, h=h):
                pltpu.sync_copy(
                    accs[h].at[pl.ds(sid * STRIPE, STRIPE)],
                    deg_out.at[out_row, 0, pl.ds(sid * STRIPE, STRIPE)],
                )


@functools.partial(
    pl.kernel,
    out_type=jax.ShapeDtypeStruct((NE, NP, 128), jnp.float32),
    mesh=_mesh,
    compiler_params=pltpu.CompilerParams(use_tc_tiling_on_sc=False),
    scratch_types=dict(
        acc=pltpu.VMEM_SHARED((NP, 32), jnp.float32),
        sidxb=pltpu.VMEM((10, 256), jnp.int32),
        didxb=pltpu.VMEM((10, 256), jnp.int32),
        sidxc=pltpu.VMEM((10, 256), jnp.int32),
        didxc=pltpu.VMEM((10, 256), jnp.int32),
        rows0=pltpu.VMEM((256, 32), jnp.float32),
        rows1=pltpu.VMEM((256, 32), jnp.float32),
        zero_v=pltpu.VMEM((56, 32), jnp.float32),
        g0=pltpu.SemaphoreType.DMA,
        g1=pltpu.SemaphoreType.DMA,
        s0=pltpu.SemaphoreType.DMA,
        s1=pltpu.SemaphoreType.DMA,
        isem=pltpu.SemaphoreType.DMA,
    ),
)
def _agg1_kernel(y, srcs, dsts, out, acc, sidxb, didxb, sidxc, didxc,
                 rows0, rows1, zero_v, g0, g1, s0, s1, isem):
    cid = lax.axis_index("c")
    sid = lax.axis_index("s")
    _fill_2d(zero_v, 56, 32, 0.0)
    r0 = sid * 40
    for e in range(NE):
        for half in range(2):
            for core in range(2):
                c = core * 2 + half

                @pl.when(cid == core)
                def _(e=e, c=c):
                    for r in range(STRIPE // 56):
                        pltpu.async_copy(
                            zero_v,
                            acc.at[pl.ds(sid * STRIPE + r * 56, 56)], s0,
                        )
                    for r in range(STRIPE // 56):
                        pltpu.make_async_copy(
                            zero_v,
                            acc.at[pl.ds(sid * STRIPE + r * 56, 56)], s0,
                        ).wait()
                    plsc.subcore_barrier()

                    bufs = [(sidxb, didxb), (sidxc, didxc)]
                    pltpu.sync_copy(srcs.at[c, e, pl.ds(r0, 10)], sidxb)
                    pltpu.sync_copy(dsts.at[e, pl.ds(r0, 10)], didxb)
                    for q in range(4):
                        sb, db = bufs[q % 2]
                        if q < 3:
                            nsb, ndb = bufs[(q + 1) % 2]
                            pltpu.async_copy(
                                srcs.at[c, e, pl.ds(r0 + (q + 1) * 10, 10)],
                                nsb, isem)
                            pltpu.async_copy(
                                dsts.at[e, pl.ds(r0 + (q + 1) * 10, 10)],
                                ndb, isem)
                        _gs_pipeline(y.at[e], sb, db, rows0, rows1,
                                     acc, g0, g1, s0, s1, 10)
                        if q < 3:
                            pltpu.make_async_copy(
                                srcs.at[c, e, pl.ds(r0 + (q + 1) * 10, 10)],
                                nsb, isem).wait()
                            pltpu.make_async_copy(
                                dsts.at[e, pl.ds(r0 + (q + 1) * 10, 10)],
                                ndb, isem).wait()
                    plsc.subcore_barrier()
                    pltpu.sync_copy(
                        acc.at[pl.ds(sid * STRIPE, STRIPE)],
                        out.at[e, pl.ds(sid * STRIPE, STRIPE),
                               pl.ds(c * 32, 32)],
                    )
                    plsc.subcore_barrier()


@functools.partial(
    pl.kernel,
    out_type=jax.ShapeDtypeStruct((2, NP, 128), jnp.float32),
    mesh=_mesh,
    compiler_params=pltpu.CompilerParams(use_tc_tiling_on_sc=False),
    scratch_types=dict(
        acc=pltpu.VMEM_SHARED((NP, 16), jnp.float32),
        sidxb=pltpu.VMEM((20, 256), jnp.int32),
        didxb=pltpu.VMEM((20, 256), jnp.int32),
        rows0=pltpu.VMEM((256, 16), jnp.float32),
        rows1=pltpu.VMEM((256, 16), jnp.float32),
        zero_v=pltpu.VMEM((196, 16), jnp.float32),
        g0=pltpu.SemaphoreType.DMA,
        g1=pltpu.SemaphoreType.DMA,
        s0=pltpu.SemaphoreType.DMA,
        s1=pltpu.SemaphoreType.DMA,
    ),
)
def _agg2_kernel(g, srcs, dsts, out, acc, sidxb, didxb, rows0, rows1, zero_v,
                 g0, g1, s0, s1):
    cid = lax.axis_index("c")
    sid = lax.axis_index("s")
    _fill_2d(zero_v, 196, 16, 0.0)
    r0 = cid * (ER2 // 2) + sid * 20
    for e in range(NE):
        for r in range(STRIPE // 196):
            pltpu.async_copy(
                zero_v, acc.at[pl.ds(sid * STRIPE + r * 196, 196)], s0
            )
        for r in range(STRIPE // 196):
            pltpu.make_async_copy(
                zero_v, acc.at[pl.ds(sid * STRIPE + r * 196, 196)], s0
            ).wait()
        plsc.subcore_barrier()
        pltpu.sync_copy(srcs.at[e, pl.ds(r0, 20)], sidxb)
        pltpu.sync_copy(dsts.at[e, pl.ds(r0, 20)], didxb)
        _gs_pipeline(g, sidxb, didxb, rows0, rows1, acc,
                     g0, g1, s0, s1, 20)
        plsc.subcore_barrier()
        pltpu.sync_copy(
            acc.at[pl.ds(sid * STRIPE, STRIPE)],
            out.at[cid, pl.ds(sid * STRIPE, STRIPE),
                   pl.ds(e * D_OUT, D_OUT)],
        )
        plsc.subcore_barrier()


def _mm1_body(x_ref, w_ref, deg_ref, out_ref):
    y = jnp.dot(x_ref[...], w_ref[...], preferred_element_type=jnp.float32)
    for e in range(NE):
        s = lax.rsqrt(jnp.maximum(deg_ref[2 * e, 0], 1.0))
        out_ref[e] = y[:, e * D_H:(e + 1) * D_H] * s[:, None]


def _mm1(x, w1cat, deg):
    return pl.pallas_call(
        _mm1_body,
        grid=(NBLK,),
        in_specs=[
            pl.BlockSpec((BN, D_IN), lambda i: (i, 0)),
            pl.BlockSpec((D_IN, NE * D_H), lambda i: (0, 0)),
            pl.BlockSpec((6, 1, BN), lambda i: (0, 0, i)),
        ],
        out_specs=pl.BlockSpec((NE, BN, 128), lambda i: (0, i, 0)),
        out_shape=jax.ShapeDtypeStruct((NE, NP, 128), jnp.float32),
    )(x, w1cat, deg)


def _mid_body(agg_ref, deg_ref, b1_ref, w2_ref, g_ref):
    i = pl.program_id(0)
    b1sum = b1_ref[0] + b1_ref[1] + b1_ref[2]
    row = i * BN + lax.broadcasted_iota(jnp.int32, (BN, 1), 0)
    h = jnp.zeros((BN, D_H), jnp.float32)
    for e in range(NE):
        s_in = lax.rsqrt(jnp.maximum(deg_ref[2 * e + 1, 0], 1.0))
        h = h + agg_ref[e] * s_in[:, None]
    h = jnp.maximum(h + b1sum[None, :], 0.0)
    h = jnp.where(row < N, h, 0.0)
    gs = []
    for e in range(NE):
        s_out = lax.rsqrt(jnp.maximum(deg_ref[2 * e, 0], 1.0))
        g = jnp.dot(h, w2_ref[e], preferred_element_type=jnp.float32)
        gs.append(g * s_out[:, None])
    gs.append(jnp.zeros((BN, 128 - NE * D_OUT), jnp.float32))
    g_ref[...] = jnp.concatenate(gs, axis=1)


def _mid(agg, deg, b1s, w2s):
    return pl.pallas_call(
        _mid_body,
        grid=(NBLK,),
        in_specs=[
            pl.BlockSpec((NE, BN, 128), lambda i: (0, i, 0)),
            pl.BlockSpec((6, 1, BN), lambda i: (0, 0, i)),
            pl.BlockSpec((NE, D_H), lambda i: (0, 0)),
            pl.BlockSpec((NE, D_H, D_OUT), lambda i: (0, 0, 0)),
        ],
        out_specs=pl.BlockSpec((BN, 128), lambda i: (i, 0)),
        out_shape=jax.ShapeDtypeStruct((NP, 128), jnp.float32),
    )(agg, deg, b1s, w2s)


def _fin_body(part_ref, deg_ref, b2_ref, out_ref):
    b2sum = b2_ref[0] + b2_ref[1] + b2_ref[2]
    p = part_ref[0] + part_ref[1]
    o = jnp.zeros((BN, D_OUT), jnp.float32)
    for e in range(NE):
        s_in = lax.rsqrt(jnp.maximum(deg_ref[2 * e + 1, 0], 1.0))
        o = o + p[:, e * D_OUT:(e + 1) * D_OUT] * s_in[:, None]
    out_ref[...] = o + b2sum[None, :]


def _fin(part, deg, b2s):
    return pl.pallas_call(
        _fin_body,
        grid=(NBLK,),
        in_specs=[
            pl.BlockSpec((2, BN, 128), lambda i: (0, i, 0)),
            pl.BlockSpec((6, 1, BN), lambda i: (0, 0, i)),
            pl.BlockSpec((NE, D_OUT), lambda i: (0, 0)),
        ],
        out_specs=pl.BlockSpec((BN, D_OUT), lambda i: (i, 0)),
        out_shape=jax.ShapeDtypeStruct((N, D_OUT), jnp.float32),
    )(part, deg, b2s)


@jax.jit
def kernel(x, edge_index_0, edge_index_1, edge_index_2,
           W1_0, W1_1, W1_2, b1_0, b1_1, b1_2,
           W2_0, W2_1, W2_2, b2_0, b2_1, b2_2):
    pad = N + (jnp.arange(EPAD, dtype=jnp.int32) % PADR)
    srcs, dsts = [], []
    for ei in (edge_index_0, edge_index_1, edge_index_2):
        e32 = ei.astype(jnp.int32)
        srcs.append(jnp.concatenate([e32[0], pad]))
        dsts.append(jnp.concatenate([e32[1], pad]))
    srcs = jnp.stack(srcs)
    dsts = jnp.stack(dsts)
    srcs2d = srcs.reshape(NE, ER2, 256)
    dsts2d = dsts.reshape(NE, ER2, 256)
    srcs4 = jnp.stack([srcs2d * 4 + c for c in range(4)])
    srcs8 = jnp.stack([srcs2d[e] * 8 + e for e in range(NE)])
    w1cat = jnp.concatenate([W1_0, W1_1, W1_2], axis=1)
    b1s = jnp.stack([b1_0, b1_1, b1_2])
    w2s = jnp.stack([W2_0, W2_1, W2_2])
    b2s = jnp.stack([b2_0, b2_1, b2_2])

    deg3 = _deg_kernel(srcs, dsts)
    y = _mm1(x, w1cat, deg3)
    y4 = y.reshape(NE, NP * 4, 32)
    agg = _agg1_kernel(y4, srcs4, dsts2d)
    g = _mid(agg, deg3, b1s, w2s)
    g16 = g.reshape(NP * 8, D_OUT)
    part = _agg2_kernel(g16, srcs8, dsts2d)
    out = _fin(part, deg3, b2s)
    return out
